# baseline (speedup 1.0000x reference)
import functools

import jax
import jax.numpy as jnp
from jax import lax
from jax.experimental import pallas as pl
from jax.experimental.pallas import tpu as pltpu
from jax.experimental.pallas import tpu_sc as plsc

NC, NS, LANES = 2, 16, 16
NW = NC * NS
D = 64
NUSERS = 100000
NITEMS = 100000
NCTX = 50000
NINST = 500000
UPAD = 102400
CPAD = 51200
IPAD = 524288
CH = 25600
SENT = 1 << 30

_mesh = plsc.VectorSubcoreMesh(
    core_axis_name="c", subcore_axis_name="s", num_cores=NC, num_subcores=NS)


def _wid():
    return lax.axis_index("s") * NC + lax.axis_index("c")


def _make_pool4(rows_total, chunk, n_gather, has_linear):
    iters = rows_total // (NW * chunk)
    per_w = rows_total // NW
    nsem = n_gather + (1 if has_linear else 0)
    scratch = []
    scratch += [pltpu.VMEM((chunk,), jnp.int32) for _ in range(n_gather)]
    scratch += [pltpu.VMEM((chunk, D), jnp.float32) for _ in range(n_gather)]
    if has_linear:
        scratch.append(pltpu.VMEM((chunk, D), jnp.float32))
    scratch.append(pltpu.VMEM((chunk, D), jnp.float32))
    scratch += [pltpu.SemaphoreType.DMA for _ in range(nsem)]

    @functools.partial(
        pl.kernel,
        out_type=jax.ShapeDtypeStruct((rows_total, D), jnp.float32),
        mesh=_mesh,
        scratch_types=scratch,
        compiler_params=pltpu.CompilerParams(use_tc_tiling_on_sc=False, needs_layout_passes=False),
    )
    def kern(*refs):
        pos = 0
        lin = None
        if has_linear:
            lin = refs[pos]; pos += 1
        idx_hbm = refs[pos:pos + n_gather]; pos += n_gather
        tab_hbm = refs[pos:pos + n_gather]; pos += n_gather
        out = refs[pos]; pos += 1
        idxb = refs[pos:pos + n_gather]; pos += n_gather
        rowb = refs[pos:pos + n_gather]; pos += n_gather
        linb = None
        if has_linear:
            linb = refs[pos]; pos += 1
        outb = refs[pos]; pos += 1
        sems = refs[pos:pos + nsem]

        w = _wid()

        @pl.loop(0, iters)
        def _chunk(it):
            base = w * per_w + it * chunk
            cps = [pltpu.async_copy(idx_hbm[j].at[pl.ds(base, chunk)],
                                    idxb[j], sems[j])
                   for j in range(n_gather)]
            if has_linear:
                cpl = pltpu.async_copy(lin.at[pl.ds(base, chunk)], linb,
                                       sems[n_gather])
            for cp in cps:
                cp.wait()
            gps = [pltpu.async_copy(tab_hbm[j].at[idxb[j]], rowb[j], sems[j])
                   for j in range(n_gather)]
            if has_linear:
                cpl.wait()
            for gp in gps:
                gp.wait()

            @pl.loop(0, chunk)
            def _row(r):
                for k in range(D // LANES):
                    sl = pl.ds(k * LANES, LANES)
                    v = rowb[0][r, sl]
                    for j in range(1, n_gather):
                        v = v + rowb[j][r, sl]
                    if has_linear:
                        v = v + linb[r, sl]
                    outb[r, sl] = v * 0.25

            pltpu.sync_copy(outb, out.at[pl.ds(base, chunk)])

    return kern


_pool_ctx = _make_pool4(CPAD, 320, 4, has_linear=False)
_pool_enc = _make_pool4(UPAD, 320, 3, has_linear=True)


IC = 2048
GR = 128
PER_TILE = IPAD // NS
ROWS_PER_TILE = CH // NS
ZB = 64


@functools.partial(
    pl.kernel,
    out_type=jax.ShapeDtypeStruct((UPAD, D), jnp.float32),
    mesh=_mesh,
    compiler_params=pltpu.CompilerParams(use_tc_tiling_on_sc=False, needs_layout_passes=False),
    scratch_types=[
        pltpu.VMEM((IC + 144,), jnp.int32),
        pltpu.VMEM((IC + 144,), jnp.int32),
        pltpu.VMEM((IC + 144,), jnp.int32),
        pltpu.VMEM((GR,), jnp.int32),
        pltpu.VMEM((GR, D), jnp.float32),
        pltpu.VMEM((GR, D), jnp.float32),
        pltpu.VMEM((ZB, D), jnp.float32),
        pltpu.VMEM_SHARED((CH + LANES, D), jnp.float32),
        pltpu.SemaphoreType.DMA,
        pltpu.SemaphoreType.DMA,
        pltpu.SemaphoreType.DMA,
        pltpu.SemaphoreType.DMA,
    ],
)
def _seg(dst_hbm, src_hbm, ctx_hbm, x_hbm, c_hbm, out,
         mdst, msrc, mctx, gd, rowX, rowC, zbuf, acc,
         semgx, semgc, semsx, semsc):
    c = lax.axis_index("c")
    s = lax.axis_index("s")

    @pl.loop(0, ZB)
    def _z(r):
        for k in range(D // LANES):
            zbuf[r, pl.ds(k * LANES, LANES)] = jnp.zeros((LANES,), jnp.float32)

    for p in range(2):
        chunk_id = 2 * p + c
        lo = chunk_id * CH

        for t in range(ROWS_PER_TILE // ZB):
            pltpu.sync_copy(zbuf, acc.at[pl.ds(s * ROWS_PER_TILE + t * ZB, ZB)])

        @pl.when(s == 0)
        def _zt():
            pltpu.sync_copy(zbuf.at[pl.ds(0, LANES)], acc.at[pl.ds(CH, LANES)])

        plsc.subcore_barrier()

        @pl.loop(0, PER_TILE // IC)
        def _scan(it):
            base = s * PER_TILE + it * IC
            pltpu.sync_copy(dst_hbm.at[pl.ds(base, IC)], mdst.at[pl.ds(0, IC)])
            pltpu.sync_copy(src_hbm.at[pl.ds(base, IC)], msrc.at[pl.ds(0, IC)])
            pltpu.sync_copy(ctx_hbm.at[pl.ds(base, IC)], mctx.at[pl.ds(0, IC)])

            def scan_body(g, ptr):
                sl = pl.ds(g * LANES, LANES)
                dv = mdst[sl]
                m = (dv >= lo) & (dv < lo + CH)
                cs = plsc.cumsum(m.astype(jnp.int32))
                pos = ptr + cs - 1
                plsc.store_scatter(mdst, [pos], dv - lo, mask=m)
                plsc.store_scatter(msrc, [pos], msrc[sl], mask=m)
                plsc.store_scatter(mctx, [pos], mctx[sl], mask=m)
                return ptr + plsc.all_reduce_population_count(m)[0]

            ptr = lax.fori_loop(0, IC // LANES, scan_body, jnp.int32(0),
                                unroll=4)

            tvec = jnp.int32(CH) + lax.iota(jnp.int32, LANES)
            zv = jnp.zeros((LANES,), jnp.int32)
            for k in range(GR // LANES):
                mdst[pl.ds(ptr + k * LANES, LANES)] = tvec
                msrc[pl.ds(ptr + k * LANES, LANES)] = zv
                mctx[pl.ds(ptr + k * LANES, LANES)] = zv

            ng = (ptr + (GR - 1)) // GR
            for k in range(IC // GR):
                @pl.when(k < ng)
                def _gran():
                    gX = pltpu.async_copy(
                        x_hbm.at[msrc.at[pl.ds(k * GR, GR)]], rowX, semgx)
                    gC = pltpu.async_copy(
                        c_hbm.at[mctx.at[pl.ds(k * GR, GR)]], rowC, semgc)
                    for v in range(GR // LANES):
                        gd[pl.ds(v * LANES, LANES)] = (
                            mdst[pl.ds(k * GR + v * LANES, LANES)])
                    gX.wait()
                    gC.wait()

                    @pl.loop(0, GR)
                    def _add(r):
                        for v in range(D // LANES):
                            sl = pl.ds(v * LANES, LANES)
                            rowX[r, sl] = rowX[r, sl] + rowC[r, sl]

                    pltpu.sync_copy(rowX, acc.at[gd], add=True)

        plsc.subcore_barrier()

        for t in range(ROWS_PER_TILE // ZB):
            r0 = s * ROWS_PER_TILE + t * ZB
            pltpu.sync_copy(acc.at[pl.ds(r0, ZB)], out.at[pl.ds(lo + r0, ZB)])

        plsc.subcore_barrier()


MS = 103424
MZC = 6464
BPT = 4096 // NS


@functools.partial(
    pl.kernel,
    out_type=jax.ShapeDtypeStruct((2, MS), jnp.int32),
    mesh=_mesh,
    compiler_params=pltpu.CompilerParams(use_tc_tiling_on_sc=False, needs_layout_passes=False),
    scratch_types=[
        pltpu.VMEM((MZC // 4,), jnp.int32),
        pltpu.VMEM((BPT,), jnp.int32),
        pltpu.VMEM((BPT,), jnp.int32),
    ],
)
def _slots(ids2_hbm, maps, zb, idb, valb):
    c = lax.axis_index("c")
    s = lax.axis_index("s")

    @pl.loop(0, MZC // 4 // LANES)
    def _zf(g):
        zb[pl.ds(g * LANES, LANES)] = jnp.zeros((LANES,), jnp.int32)

    @pl.loop(0, BPT // LANES)
    def _vf(g):
        valb[pl.ds(g * LANES, LANES)] = (
            s * BPT + g * LANES + 1 + lax.iota(jnp.int32, LANES))

    for t in range(4):
        pltpu.sync_copy(zb, maps.at[c].at[pl.ds(s * MZC + t * (MZC // 4),
                                                MZC // 4)])
    plsc.subcore_barrier()
    pltpu.sync_copy(ids2_hbm.at[c].at[pl.ds(s * BPT, BPT)], idb)
    pltpu.sync_copy(valb, maps.at[c].at[idb])


NSLOT = 4096
IC2 = 2048
PT2 = IPAD // NW


@functools.partial(
    pl.kernel,
    out_type=(jax.ShapeDtypeStruct((2 * NSLOT, D), jnp.float32),
              jax.ShapeDtypeStruct((2 * NSLOT, D), jnp.float32)),
    mesh=_mesh,
    compiler_params=pltpu.CompilerParams(use_tc_tiling_on_sc=False, needs_layout_passes=False),
    scratch_types=[
        pltpu.VMEM((IC2,), jnp.int32),
        pltpu.VMEM((IC2,), jnp.int32),
        pltpu.VMEM((IC2,), jnp.int32),
        pltpu.VMEM((IC2,), jnp.int32),
        pltpu.VMEM((IC2,), jnp.int32),
        pltpu.VMEM((IC2 + 144,), jnp.int32),
        pltpu.VMEM((IC2 + 144,), jnp.int32),
        pltpu.VMEM((IC2 + 144,), jnp.int32),
        pltpu.VMEM((IC2 + 144,), jnp.int32),
        pltpu.VMEM((IC2 + 144,), jnp.int32),
        pltpu.VMEM((IC2 + 144,), jnp.int32),
        pltpu.VMEM((GR,), jnp.int32),
        pltpu.VMEM((GR, D), jnp.float32),
        pltpu.VMEM((GR, D), jnp.float32),
        pltpu.VMEM((ZB, D), jnp.float32),
        pltpu.VMEM_SHARED((NSLOT + LANES, D), jnp.float32),
        pltpu.VMEM_SHARED((NSLOT + LANES, D), jnp.float32),
        pltpu.SemaphoreType.DMA,
        pltpu.SemaphoreType.DMA,
        pltpu.SemaphoreType.DMA,
        pltpu.SemaphoreType.DMA,
    ],
)
def _seg2(uid_hbm, iid_hbm, cid_hbm, mapu_hbm, mapi_hbm,
          u1_hbm, i1_hbm, c_hbm, out_u, out_i,
          ubuf, ibuf, cbuf, su, si,
          mu_dst, mu_src, mu_ctx, mi_dst, mi_src, mi_ctx,
          gd, rowX, rowC, zbuf, acc_u, acc_i,
          semgx, semgc, semsx, semsc):
    c = lax.axis_index("c")
    s = lax.axis_index("s")

    @pl.loop(0, ZB)
    def _z(r):
        for k in range(D // LANES):
            zbuf[r, pl.ds(k * LANES, LANES)] = jnp.zeros((LANES,), jnp.float32)

    rpt = NSLOT // NS
    for acc in (acc_u, acc_i):
        for t in range(rpt // ZB):
            pltpu.sync_copy(zbuf, acc.at[pl.ds(s * rpt + t * ZB, ZB)])

        @pl.when(s == 0)
        def _zt():
            pltpu.sync_copy(zbuf.at[pl.ds(0, LANES)],
                            acc.at[pl.ds(NSLOT, LANES)])

    plsc.subcore_barrier()

    @pl.loop(0, PT2 // IC2)
    def _scan(it):
        base = (c * NS + s) * PT2 + it * IC2
        pltpu.sync_copy(uid_hbm.at[pl.ds(base, IC2)], ubuf)
        pltpu.sync_copy(iid_hbm.at[pl.ds(base, IC2)], ibuf)
        pltpu.sync_copy(cid_hbm.at[pl.ds(base, IC2)], cbuf)
        pltpu.sync_copy(mapu_hbm.at[ubuf], su)
        pltpu.sync_copy(mapi_hbm.at[ibuf], si)

        def scan_body(g, carry):
            pu, pi = carry
            sl = pl.ds(g * LANES, LANES)
            sv = su[sl]
            mu = sv > 0
            csu = plsc.cumsum(mu.astype(jnp.int32))
            posu = pu + csu - 1
            plsc.store_scatter(mu_dst, [posu], sv - 1, mask=mu)
            plsc.store_scatter(mu_src, [posu], ibuf[sl], mask=mu)
            plsc.store_scatter(mu_ctx, [posu], cbuf[sl], mask=mu)
            tv = si[sl]
            mi = tv > 0
            csi = plsc.cumsum(mi.astype(jnp.int32))
            posi = pi + csi - 1
            plsc.store_scatter(mi_dst, [posi], tv - 1, mask=mi)
            plsc.store_scatter(mi_src, [posi], ubuf[sl], mask=mi)
            plsc.store_scatter(mi_ctx, [posi], cbuf[sl], mask=mi)
            return (pu + plsc.all_reduce_population_count(mu)[0],
                    pi + plsc.all_reduce_population_count(mi)[0])

        pu, pi = lax.fori_loop(0, IC2 // LANES, scan_body,
                               (jnp.int32(0), jnp.int32(0)), unroll=4)

        tvec = jnp.int32(NSLOT) + lax.iota(jnp.int32, LANES)
        zv = jnp.zeros((LANES,), jnp.int32)
        for k in range(GR // LANES):
            mu_dst[pl.ds(pu + k * LANES, LANES)] = tvec
            mu_src[pl.ds(pu + k * LANES, LANES)] = zv
            mu_ctx[pl.ds(pu + k * LANES, LANES)] = zv
            mi_dst[pl.ds(pi + k * LANES, LANES)] = tvec
            mi_src[pl.ds(pi + k * LANES, LANES)] = zv
            mi_ctx[pl.ds(pi + k * LANES, LANES)] = zv

        for (md, msrc_, mc, xh, accr, ptr) in (
                (mu_dst, mu_src, mu_ctx, i1_hbm, acc_u, pu),
                (mi_dst, mi_src, mi_ctx, u1_hbm, acc_i, pi)):
            ng = (ptr + (GR - 1)) // GR
            for k in range(IC2 // GR):
                @pl.when(k < ng)
                def _gran():
                    gX = pltpu.async_copy(
                        xh.at[msrc_.at[pl.ds(k * GR, GR)]], rowX, semgx)
                    gC = pltpu.async_copy(
                        c_hbm.at[mc.at[pl.ds(k * GR, GR)]], rowC, semgc)
                    for v in range(GR // LANES):
                        gd[pl.ds(v * LANES, LANES)] = (
                            md[pl.ds(k * GR + v * LANES, LANES)])
                    gX.wait()
                    gC.wait()

                    @pl.loop(0, GR)
                    def _add(r):
                        for v in range(D // LANES):
                            sl = pl.ds(v * LANES, LANES)
                            rowX[r, sl] = rowX[r, sl] + rowC[r, sl]

                    pltpu.sync_copy(rowX, accr.at[gd], add=True)

    plsc.subcore_barrier()

    rpt = NSLOT // NS
    for (accr, outr) in ((acc_u, out_u), (acc_i, out_i)):
        for t in range(rpt // ZB):
            r0 = s * rpt + t * ZB
            pltpu.sync_copy(accr.at[pl.ds(r0, ZB)],
                            outr.at[pl.ds(c * NSLOT + r0, ZB)])


BB = 4096 // NW


@functools.partial(
    pl.kernel,
    out_type=jax.ShapeDtypeStruct((4096,), jnp.float32),
    mesh=_mesh,
    compiler_params=pltpu.CompilerParams(use_tc_tiling_on_sc=False, needs_layout_passes=False),
    scratch_types=(
        [pltpu.VMEM((BB,), jnp.int32) for _ in range(7)]
        + [pltpu.VMEM((BB,), jnp.int32) for _ in range(6)]
        + [pltpu.VMEM((BB, D), jnp.float32) for _ in range(12)]
        + [pltpu.VMEM((BB,), jnp.float32) for _ in range(2)]
        + [pltpu.VMEM((LANES,), jnp.float32), pltpu.VMEM((BB,), jnp.float32)]
    ),
)
def _decode(uid_hbm, iid_hbm, cid_hbm, cf0, cf1, cf2, cf3,
            eu_hbm, u1_hbm, u2p_hbm, ei_hbm, i1_hbm, i2p_hbm,
            mapu_hbm, mapi_hbm,
            cfe_hbm, ie_hbm, ub_hbm, ib_hbm, gb_hbm, out,
            ub, ib, cb, cm0, cm1, cm2, cm3,
            su, si, ju0, ju1, ji0, ji1,
            bEu, bU1, bU2a, bU2b, bEi, bI1, bI2a, bI2b, r2, r3, r4, r5,
            bub, bib, bgb, outb):
    w = _wid()
    base = w * BB
    pltpu.sync_copy(uid_hbm.at[pl.ds(base, BB)], ub)
    pltpu.sync_copy(iid_hbm.at[pl.ds(base, BB)], ib)
    pltpu.sync_copy(cid_hbm.at[pl.ds(base, BB)], cb)
    pltpu.sync_copy(cf0.at[cb], cm0)
    pltpu.sync_copy(cf1.at[cb], cm1)
    pltpu.sync_copy(cf2.at[cb], cm2)
    pltpu.sync_copy(cf3.at[cb], cm3)
    pltpu.sync_copy(mapu_hbm.at[ub], su)
    pltpu.sync_copy(mapi_hbm.at[ib], si)

    @pl.loop(0, BB // LANES)
    def _ji(g):
        sl = pl.ds(g * LANES, LANES)
        vu = su[sl] - 1
        ju0[sl] = vu
        ju1[sl] = vu + NSLOT
        vi = si[sl] - 1
        ji0[sl] = vi
        ji1[sl] = vi + NSLOT

    pltpu.sync_copy(eu_hbm.at[ub], bEu)
    pltpu.sync_copy(u1_hbm.at[ub], bU1)
    pltpu.sync_copy(u2p_hbm.at[ju0], bU2a)
    pltpu.sync_copy(u2p_hbm.at[ju1], bU2b)
    pltpu.sync_copy(ei_hbm.at[ib], bEi)
    pltpu.sync_copy(i1_hbm.at[ib], bI1)
    pltpu.sync_copy(i2p_hbm.at[ji0], bI2a)
    pltpu.sync_copy(i2p_hbm.at[ji1], bI2b)
    pltpu.sync_copy(cfe_hbm.at[cm0], r2)
    pltpu.sync_copy(cfe_hbm.at[cm1], r3)
    pltpu.sync_copy(cfe_hbm.at[cm2], r4)
    pltpu.sync_copy(ie_hbm.at[cm3], r5)
    pltpu.sync_copy(ub_hbm.at[ub], bub)
    pltpu.sync_copy(ib_hbm.at[ib], bib)
    pltpu.sync_copy(gb_hbm, bgb)

    @pl.loop(0, BB // LANES)
    def _grp(g):
        r0 = g * LANES
        lanei = lax.iota(jnp.int32, LANES)
        res = jnp.zeros((LANES,), jnp.float32)
        for j in range(LANES):
            r = r0 + j
            tv = jnp.zeros((LANES,), jnp.float32)
            for k in range(D // LANES):
                sl = pl.ds(k * LANES, LANES)
                fu = (bEu[r, sl] + 0.5 * bU1[r, sl]
                      + 0.25 * (bU2a[r, sl] + bU2b[r, sl]))
                fi = (bEi[r, sl] + 0.5 * bI1[r, sl]
                      + 0.25 * (bI2a[r, sl] + bI2b[r, sl]))
                a = r2[r, sl]
                b = r3[r, sl]
                cc = r4[r, sl]
                dd = r5[r, sl]
                ssum = fu + fi + a + b + cc + dd
                sq = (fu * fu + fi * fi + a * a + b * b + cc * cc + dd * dd)
                tv = tv + (ssum * ssum - sq)
            tot = jnp.sum(tv)
            res = jnp.where(lanei == j, tot, res)
        res = (0.5 * res + bub[pl.ds(r0, LANES)] + bib[pl.ds(r0, LANES)]
               + bgb[pl.ds(0, LANES)])
        outb[pl.ds(r0, LANES)] = res

    pltpu.sync_copy(outb, out.at[pl.ds(base, BB)])


def _pad_rows(a, n):
    return jnp.concatenate(
        [a, jnp.zeros((n - a.shape[0],) + a.shape[1:], a.dtype)], axis=0)


def kernel(user_embeddings, item_embeddings, user_feature_embeddings,
           item_feature_embeddings, context_feature_embeddings,
           user_bias, item_bias, global_bias,
           user_id, item_id, context_id,
           user_feature_mat, item_feature_mat, context_feature_mat,
           insts2userid, insts2itemid, insts2contextid):
    ue_p = _pad_rows(user_embeddings, UPAD)
    ie_p = _pad_rows(item_embeddings, UPAD)
    ufm_p = _pad_rows(user_feature_mat, UPAD)
    ifm_p = _pad_rows(item_feature_mat, UPAD)
    cfm_p = _pad_rows(context_feature_mat, CPAD)
    uf = [ufm_p[:, j] + 0 for j in range(3)]
    if_ = [ifm_p[:, j] + 0 for j in range(3)]
    cf = [cfm_p[:, j] + 0 for j in range(4)]

    npad = IPAD - NINST
    uid_p = jnp.concatenate([insts2userid, jnp.full((npad,), SENT, jnp.int32)])
    iid_p = jnp.concatenate([insts2itemid, jnp.full((npad,), SENT, jnp.int32)])
    cid_p = jnp.concatenate([insts2contextid, jnp.zeros((npad,), jnp.int32)])

    C = _pool_ctx(cf[0], cf[1], cf[2], cf[3],
                  context_feature_embeddings, context_feature_embeddings,
                  context_feature_embeddings, ie_p)
    EU = _pool_enc(ue_p, uf[0], uf[1], uf[2],
                   user_feature_embeddings, user_feature_embeddings,
                   user_feature_embeddings)
    EI = _pool_enc(ie_p, if_[0], if_[1], if_[2],
                   item_feature_embeddings, item_feature_embeddings,
                   item_feature_embeddings)

    U1 = _seg(uid_p, iid_p, cid_p, EI, C)
    I1 = _seg(iid_p, uid_p, cid_p, EU, C)

    maps = _slots(jnp.stack([user_id, item_id]))
    map_u = maps[0] + 0
    map_i = maps[1] + 0
    U2P, I2P = _seg2(uid_p, iid_p, cid_p, map_u, map_i, U1, I1, C)

    ub_flat = user_bias[:, 0] + 0
    ib_flat = item_bias[:, 0] + 0
    gb16 = jnp.broadcast_to(global_bias[0, 0], (LANES,)) + 0

    pred = _decode(user_id, item_id, context_id, cf[0], cf[1], cf[2], cf[3],
                   EU, U1, U2P, EI, I1, I2P, map_u, map_i,
                   context_feature_embeddings, ie_p,
                   ub_flat, ib_flat, gb16)
    return pred.reshape(4096, 1)


# speedup vs baseline: 1.0006x; 1.0006x over previous
import functools

import jax
import jax.numpy as jnp
from jax import lax
from jax.experimental import pallas as pl
from jax.experimental.pallas import tpu as pltpu
from jax.experimental.pallas import tpu_sc as plsc

NC, NS, LANES = 2, 16, 16
NW = NC * NS
D = 64
NUSERS = 100000
NITEMS = 100000
NCTX = 50000
NINST = 500000
UPAD = 102400
CPAD = 51200
IPAD = 524288
CH = 25600
SENT = 1 << 30

_mesh = plsc.VectorSubcoreMesh(
    core_axis_name="c", subcore_axis_name="s", num_cores=NC, num_subcores=NS)


def _wid():
    return lax.axis_index("s") * NC + lax.axis_index("c")


def _make_pool4(rows_total, chunk, n_gather, has_linear):
    iters = rows_total // (NW * chunk)
    per_w = rows_total // NW
    nsem = n_gather + (1 if has_linear else 0)
    scratch = []
    scratch += [pltpu.VMEM((chunk,), jnp.int32) for _ in range(n_gather)]
    scratch += [pltpu.VMEM((chunk, D), jnp.float32) for _ in range(n_gather)]
    if has_linear:
        scratch.append(pltpu.VMEM((chunk, D), jnp.float32))
    scratch.append(pltpu.VMEM((chunk, D), jnp.float32))
    scratch += [pltpu.SemaphoreType.DMA for _ in range(nsem)]

    @functools.partial(
        pl.kernel,
        out_type=jax.ShapeDtypeStruct((rows_total, D), jnp.float32),
        mesh=_mesh,
        scratch_types=scratch,
        compiler_params=pltpu.CompilerParams(use_tc_tiling_on_sc=False, needs_layout_passes=False),
    )
    def kern(*refs):
        pos = 0
        lin = None
        if has_linear:
            lin = refs[pos]; pos += 1
        idx_hbm = refs[pos:pos + n_gather]; pos += n_gather
        tab_hbm = refs[pos:pos + n_gather]; pos += n_gather
        out = refs[pos]; pos += 1
        idxb = refs[pos:pos + n_gather]; pos += n_gather
        rowb = refs[pos:pos + n_gather]; pos += n_gather
        linb = None
        if has_linear:
            linb = refs[pos]; pos += 1
        outb = refs[pos]; pos += 1
        sems = refs[pos:pos + nsem]

        w = _wid()

        @pl.loop(0, iters)
        def _chunk(it):
            base = w * per_w + it * chunk
            cps = [pltpu.async_copy(idx_hbm[j].at[pl.ds(base, chunk)],
                                    idxb[j], sems[j])
                   for j in range(n_gather)]
            if has_linear:
                cpl = pltpu.async_copy(lin.at[pl.ds(base, chunk)], linb,
                                       sems[n_gather])
            for cp in cps:
                cp.wait()
            gps = [pltpu.async_copy(tab_hbm[j].at[idxb[j]], rowb[j], sems[j])
                   for j in range(n_gather)]
            if has_linear:
                cpl.wait()
            for gp in gps:
                gp.wait()

            @pl.loop(0, chunk)
            def _row(r):
                for k in range(D // LANES):
                    sl = pl.ds(k * LANES, LANES)
                    v = rowb[0][r, sl]
                    for j in range(1, n_gather):
                        v = v + rowb[j][r, sl]
                    if has_linear:
                        v = v + linb[r, sl]
                    outb[r, sl] = v * 0.25

            pltpu.sync_copy(outb, out.at[pl.ds(base, chunk)])

    return kern


_pool_ctx = _make_pool4(CPAD, 320, 4, has_linear=False)
_pool_enc = _make_pool4(UPAD, 320, 3, has_linear=True)


IC = 2048
GR = 128
PER_TILE = IPAD // NS
ROWS_PER_TILE = CH // NS
ZB = 64


@functools.partial(
    pl.kernel,
    out_type=jax.ShapeDtypeStruct((UPAD, D), jnp.float32),
    mesh=_mesh,
    compiler_params=pltpu.CompilerParams(use_tc_tiling_on_sc=False, needs_layout_passes=False),
    scratch_types=[
        pltpu.VMEM((IC + 144,), jnp.int32),
        pltpu.VMEM((IC + 144,), jnp.int32),
        pltpu.VMEM((IC + 144,), jnp.int32),
        pltpu.VMEM((GR,), jnp.int32),
        pltpu.VMEM((GR,), jnp.int32),
        pltpu.VMEM((GR, D), jnp.float32),
        pltpu.VMEM((GR, D), jnp.float32),
        pltpu.VMEM((ZB, D), jnp.float32),
        pltpu.VMEM_SHARED((CH + LANES, D), jnp.float32),
        pltpu.SemaphoreType.DMA,
        pltpu.SemaphoreType.DMA,
        pltpu.SemaphoreType.DMA,
        pltpu.SemaphoreType.DMA,
    ],
)
def _seg(dst_hbm, src_hbm, ctx_hbm, x_hbm, c_hbm, out,
         mdst, msrc, mctx, gd, gd2, rowX, rowC, zbuf, acc,
         semgx, semgc, semsx, semsc):
    c = lax.axis_index("c")
    s = lax.axis_index("s")

    @pl.loop(0, ZB)
    def _z(r):
        for k in range(D // LANES):
            zbuf[r, pl.ds(k * LANES, LANES)] = jnp.zeros((LANES,), jnp.float32)

    for p in range(2):
        chunk_id = 2 * p + c
        lo = chunk_id * CH

        for t in range(ROWS_PER_TILE // ZB):
            pltpu.sync_copy(zbuf, acc.at[pl.ds(s * ROWS_PER_TILE + t * ZB, ZB)])

        @pl.when(s == 0)
        def _zt():
            pltpu.sync_copy(zbuf.at[pl.ds(0, LANES)], acc.at[pl.ds(CH, LANES)])

        plsc.subcore_barrier()

        @pl.loop(0, PER_TILE // IC)
        def _scan(it):
            base = s * PER_TILE + it * IC
            pltpu.sync_copy(dst_hbm.at[pl.ds(base, IC)], mdst.at[pl.ds(0, IC)])
            pltpu.sync_copy(src_hbm.at[pl.ds(base, IC)], msrc.at[pl.ds(0, IC)])
            pltpu.sync_copy(ctx_hbm.at[pl.ds(base, IC)], mctx.at[pl.ds(0, IC)])

            def scan_body(g, ptr):
                sl = pl.ds(g * LANES, LANES)
                dv = mdst[sl]
                m = (dv >= lo) & (dv < lo + CH)
                cs = plsc.cumsum(m.astype(jnp.int32))
                pos = ptr + cs - 1
                plsc.store_scatter(mdst, [pos], dv - lo, mask=m)
                plsc.store_scatter(msrc, [pos], msrc[sl], mask=m)
                plsc.store_scatter(mctx, [pos], mctx[sl], mask=m)
                return ptr + plsc.all_reduce_population_count(m)[0]

            ptr = lax.fori_loop(0, IC // LANES, scan_body, jnp.int32(0),
                                unroll=4)

            tvec = jnp.int32(CH) + lax.iota(jnp.int32, LANES)
            zv = jnp.zeros((LANES,), jnp.int32)
            for k in range(GR // LANES):
                mdst[pl.ds(ptr + k * LANES, LANES)] = tvec
                msrc[pl.ds(ptr + k * LANES, LANES)] = zv
                mctx[pl.ds(ptr + k * LANES, LANES)] = zv

            ng = (ptr + (GR - 1)) // GR
            descs = {}
            for k in range(IC // GR):
                @pl.when(k < ng)
                def _gran(k=k):
                    gdk = gd if k % 2 == 0 else gd2
                    gX = pltpu.async_copy(
                        x_hbm.at[msrc.at[pl.ds(k * GR, GR)]], rowX, semgx)
                    for v in range(GR // LANES):
                        gdk[pl.ds(v * LANES, LANES)] = (
                            mdst[pl.ds(k * GR + v * LANES, LANES)])
                    if k > 0:
                        descs[k - 1].wait()
                    gC = pltpu.async_copy(
                        c_hbm.at[mctx.at[pl.ds(k * GR, GR)]], rowC, semgc)
                    gX.wait()
                    gC.wait()

                    @pl.loop(0, GR)
                    def _add(r):
                        for v in range(D // LANES):
                            sl = pl.ds(v * LANES, LANES)
                            rowC[r, sl] = rowX[r, sl] + rowC[r, sl]

                    descs[k] = pltpu.async_copy(rowC, acc.at[gdk], semsx,
                                                add=True)

                    @pl.when(k == ng - 1)
                    def _last():
                        descs[k].wait()

        plsc.subcore_barrier()

        for t in range(ROWS_PER_TILE // ZB):
            r0 = s * ROWS_PER_TILE + t * ZB
            pltpu.sync_copy(acc.at[pl.ds(r0, ZB)], out.at[pl.ds(lo + r0, ZB)])

        plsc.subcore_barrier()


MS = 103424
MZC = 6464
BPT = 4096 // NS


@functools.partial(
    pl.kernel,
    out_type=jax.ShapeDtypeStruct((2, MS), jnp.int32),
    mesh=_mesh,
    compiler_params=pltpu.CompilerParams(use_tc_tiling_on_sc=False, needs_layout_passes=False),
    scratch_types=[
        pltpu.VMEM((MZC // 4,), jnp.int32),
        pltpu.VMEM((BPT,), jnp.int32),
        pltpu.VMEM((BPT,), jnp.int32),
    ],
)
def _slots(ids2_hbm, maps, zb, idb, valb):
    c = lax.axis_index("c")
    s = lax.axis_index("s")

    @pl.loop(0, MZC // 4 // LANES)
    def _zf(g):
        zb[pl.ds(g * LANES, LANES)] = jnp.zeros((LANES,), jnp.int32)

    @pl.loop(0, BPT // LANES)
    def _vf(g):
        valb[pl.ds(g * LANES, LANES)] = (
            s * BPT + g * LANES + 1 + lax.iota(jnp.int32, LANES))

    for t in range(4):
        pltpu.sync_copy(zb, maps.at[c].at[pl.ds(s * MZC + t * (MZC // 4),
                                                MZC // 4)])
    plsc.subcore_barrier()
    pltpu.sync_copy(ids2_hbm.at[c].at[pl.ds(s * BPT, BPT)], idb)
    pltpu.sync_copy(valb, maps.at[c].at[idb])


NSLOT = 4096
IC2 = 2048
PT2 = IPAD // NW


@functools.partial(
    pl.kernel,
    out_type=(jax.ShapeDtypeStruct((2 * NSLOT, D), jnp.float32),
              jax.ShapeDtypeStruct((2 * NSLOT, D), jnp.float32)),
    mesh=_mesh,
    compiler_params=pltpu.CompilerParams(use_tc_tiling_on_sc=False, needs_layout_passes=False),
    scratch_types=[
        pltpu.VMEM((IC2,), jnp.int32),
        pltpu.VMEM((IC2,), jnp.int32),
        pltpu.VMEM((IC2,), jnp.int32),
        pltpu.VMEM((IC2,), jnp.int32),
        pltpu.VMEM((IC2,), jnp.int32),
        pltpu.VMEM((IC2 + 144,), jnp.int32),
        pltpu.VMEM((IC2 + 144,), jnp.int32),
        pltpu.VMEM((IC2 + 144,), jnp.int32),
        pltpu.VMEM((IC2 + 144,), jnp.int32),
        pltpu.VMEM((IC2 + 144,), jnp.int32),
        pltpu.VMEM((IC2 + 144,), jnp.int32),
        pltpu.VMEM((GR,), jnp.int32),
        pltpu.VMEM((GR,), jnp.int32),
        pltpu.VMEM((GR, D), jnp.float32),
        pltpu.VMEM((GR, D), jnp.float32),
        pltpu.VMEM((ZB, D), jnp.float32),
        pltpu.VMEM_SHARED((NSLOT + LANES, D), jnp.float32),
        pltpu.VMEM_SHARED((NSLOT + LANES, D), jnp.float32),
        pltpu.SemaphoreType.DMA,
        pltpu.SemaphoreType.DMA,
        pltpu.SemaphoreType.DMA,
        pltpu.SemaphoreType.DMA,
    ],
)
def _seg2(uid_hbm, iid_hbm, cid_hbm, mapu_hbm, mapi_hbm,
          u1_hbm, i1_hbm, c_hbm, out_u, out_i,
          ubuf, ibuf, cbuf, su, si,
          mu_dst, mu_src, mu_ctx, mi_dst, mi_src, mi_ctx,
          gd, gd2, rowX, rowC, zbuf, acc_u, acc_i,
          semgx, semgc, semsx, semsc):
    c = lax.axis_index("c")
    s = lax.axis_index("s")

    @pl.loop(0, ZB)
    def _z(r):
        for k in range(D // LANES):
            zbuf[r, pl.ds(k * LANES, LANES)] = jnp.zeros((LANES,), jnp.float32)

    rpt = NSLOT // NS
    for acc in (acc_u, acc_i):
        for t in range(rpt // ZB):
            pltpu.sync_copy(zbuf, acc.at[pl.ds(s * rpt + t * ZB, ZB)])

        @pl.when(s == 0)
        def _zt():
            pltpu.sync_copy(zbuf.at[pl.ds(0, LANES)],
                            acc.at[pl.ds(NSLOT, LANES)])

    plsc.subcore_barrier()

    @pl.loop(0, PT2 // IC2)
    def _scan(it):
        base = (c * NS + s) * PT2 + it * IC2
        pltpu.sync_copy(uid_hbm.at[pl.ds(base, IC2)], ubuf)
        pltpu.sync_copy(iid_hbm.at[pl.ds(base, IC2)], ibuf)
        pltpu.sync_copy(cid_hbm.at[pl.ds(base, IC2)], cbuf)
        pltpu.sync_copy(mapu_hbm.at[ubuf], su)
        pltpu.sync_copy(mapi_hbm.at[ibuf], si)

        def scan_body(g, carry):
            pu, pi = carry
            sl = pl.ds(g * LANES, LANES)
            sv = su[sl]
            mu = sv > 0
            csu = plsc.cumsum(mu.astype(jnp.int32))
            posu = pu + csu - 1
            plsc.store_scatter(mu_dst, [posu], sv - 1, mask=mu)
            plsc.store_scatter(mu_src, [posu], ibuf[sl], mask=mu)
            plsc.store_scatter(mu_ctx, [posu], cbuf[sl], mask=mu)
            tv = si[sl]
            mi = tv > 0
            csi = plsc.cumsum(mi.astype(jnp.int32))
            posi = pi + csi - 1
            plsc.store_scatter(mi_dst, [posi], tv - 1, mask=mi)
            plsc.store_scatter(mi_src, [posi], ubuf[sl], mask=mi)
            plsc.store_scatter(mi_ctx, [posi], cbuf[sl], mask=mi)
            return (pu + plsc.all_reduce_population_count(mu)[0],
                    pi + plsc.all_reduce_population_count(mi)[0])

        pu, pi = lax.fori_loop(0, IC2 // LANES, scan_body,
                               (jnp.int32(0), jnp.int32(0)), unroll=4)

        tvec = jnp.int32(NSLOT) + lax.iota(jnp.int32, LANES)
        zv = jnp.zeros((LANES,), jnp.int32)
        for k in range(GR // LANES):
            mu_dst[pl.ds(pu + k * LANES, LANES)] = tvec
            mu_src[pl.ds(pu + k * LANES, LANES)] = zv
            mu_ctx[pl.ds(pu + k * LANES, LANES)] = zv
            mi_dst[pl.ds(pi + k * LANES, LANES)] = tvec
            mi_src[pl.ds(pi + k * LANES, LANES)] = zv
            mi_ctx[pl.ds(pi + k * LANES, LANES)] = zv

        for (md, msrc_, mc, xh, accr, ptr) in (
                (mu_dst, mu_src, mu_ctx, i1_hbm, acc_u, pu),
                (mi_dst, mi_src, mi_ctx, u1_hbm, acc_i, pi)):
            ng = (ptr + (GR - 1)) // GR
            descs = {}
            for k in range(IC2 // GR):
                @pl.when(k < ng)
                def _gran(k=k, md=md, msrc_=msrc_, mc=mc, xh=xh, accr=accr):
                    gdk = gd if k % 2 == 0 else gd2
                    gX = pltpu.async_copy(
                        xh.at[msrc_.at[pl.ds(k * GR, GR)]], rowX, semgx)
                    for v in range(GR // LANES):
                        gdk[pl.ds(v * LANES, LANES)] = (
                            md[pl.ds(k * GR + v * LANES, LANES)])
                    if k > 0:
                        descs[k - 1].wait()
                    gC = pltpu.async_copy(
                        c_hbm.at[mc.at[pl.ds(k * GR, GR)]], rowC, semgc)
                    gX.wait()
                    gC.wait()

                    @pl.loop(0, GR)
                    def _add(r):
                        for v in range(D // LANES):
                            sl = pl.ds(v * LANES, LANES)
                            rowC[r, sl] = rowX[r, sl] + rowC[r, sl]

                    descs[k] = pltpu.async_copy(rowC, accr.at[gdk], semsx,
                                                add=True)

                    @pl.when(k == ng - 1)
                    def _last():
                        descs[k].wait()

    plsc.subcore_barrier()

    rpt = NSLOT // NS
    for (accr, outr) in ((acc_u, out_u), (acc_i, out_i)):
        for t in range(rpt // ZB):
            r0 = s * rpt + t * ZB
            pltpu.sync_copy(accr.at[pl.ds(r0, ZB)],
                            outr.at[pl.ds(c * NSLOT + r0, ZB)])


BB = 4096 // NW


@functools.partial(
    pl.kernel,
    out_type=jax.ShapeDtypeStruct((4096,), jnp.float32),
    mesh=_mesh,
    compiler_params=pltpu.CompilerParams(use_tc_tiling_on_sc=False, needs_layout_passes=False),
    scratch_types=(
        [pltpu.VMEM((BB,), jnp.int32) for _ in range(7)]
        + [pltpu.VMEM((BB,), jnp.int32) for _ in range(6)]
        + [pltpu.VMEM((BB, D), jnp.float32) for _ in range(12)]
        + [pltpu.VMEM((BB,), jnp.float32) for _ in range(2)]
        + [pltpu.VMEM((LANES,), jnp.float32), pltpu.VMEM((BB,), jnp.float32)]
    ),
)
def _decode(uid_hbm, iid_hbm, cid_hbm, cf0, cf1, cf2, cf3,
            eu_hbm, u1_hbm, u2p_hbm, ei_hbm, i1_hbm, i2p_hbm,
            mapu_hbm, mapi_hbm,
            cfe_hbm, ie_hbm, ub_hbm, ib_hbm, gb_hbm, out,
            ub, ib, cb, cm0, cm1, cm2, cm3,
            su, si, ju0, ju1, ji0, ji1,
            bEu, bU1, bU2a, bU2b, bEi, bI1, bI2a, bI2b, r2, r3, r4, r5,
            bub, bib, bgb, outb):
    w = _wid()
    base = w * BB
    pltpu.sync_copy(uid_hbm.at[pl.ds(base, BB)], ub)
    pltpu.sync_copy(iid_hbm.at[pl.ds(base, BB)], ib)
    pltpu.sync_copy(cid_hbm.at[pl.ds(base, BB)], cb)
    pltpu.sync_copy(cf0.at[cb], cm0)
    pltpu.sync_copy(cf1.at[cb], cm1)
    pltpu.sync_copy(cf2.at[cb], cm2)
    pltpu.sync_copy(cf3.at[cb], cm3)
    pltpu.sync_copy(mapu_hbm.at[ub], su)
    pltpu.sync_copy(mapi_hbm.at[ib], si)

    @pl.loop(0, BB // LANES)
    def _ji(g):
        sl = pl.ds(g * LANES, LANES)
        vu = su[sl] - 1
        ju0[sl] = vu
        ju1[sl] = vu + NSLOT
        vi = si[sl] - 1
        ji0[sl] = vi
        ji1[sl] = vi + NSLOT

    pltpu.sync_copy(eu_hbm.at[ub], bEu)
    pltpu.sync_copy(u1_hbm.at[ub], bU1)
    pltpu.sync_copy(u2p_hbm.at[ju0], bU2a)
    pltpu.sync_copy(u2p_hbm.at[ju1], bU2b)
    pltpu.sync_copy(ei_hbm.at[ib], bEi)
    pltpu.sync_copy(i1_hbm.at[ib], bI1)
    pltpu.sync_copy(i2p_hbm.at[ji0], bI2a)
    pltpu.sync_copy(i2p_hbm.at[ji1], bI2b)
    pltpu.sync_copy(cfe_hbm.at[cm0], r2)
    pltpu.sync_copy(cfe_hbm.at[cm1], r3)
    pltpu.sync_copy(cfe_hbm.at[cm2], r4)
    pltpu.sync_copy(ie_hbm.at[cm3], r5)
    pltpu.sync_copy(ub_hbm.at[ub], bub)
    pltpu.sync_copy(ib_hbm.at[ib], bib)
    pltpu.sync_copy(gb_hbm, bgb)

    @pl.loop(0, BB // LANES)
    def _grp(g):
        r0 = g * LANES
        lanei = lax.iota(jnp.int32, LANES)
        res = jnp.zeros((LANES,), jnp.float32)
        for j in range(LANES):
            r = r0 + j
            tv = jnp.zeros((LANES,), jnp.float32)
            for k in range(D // LANES):
                sl = pl.ds(k * LANES, LANES)
                fu = (bEu[r, sl] + 0.5 * bU1[r, sl]
                      + 0.25 * (bU2a[r, sl] + bU2b[r, sl]))
                fi = (bEi[r, sl] + 0.5 * bI1[r, sl]
                      + 0.25 * (bI2a[r, sl] + bI2b[r, sl]))
                a = r2[r, sl]
                b = r3[r, sl]
                cc = r4[r, sl]
                dd = r5[r, sl]
                ssum = fu + fi + a + b + cc + dd
                sq = (fu * fu + fi * fi + a * a + b * b + cc * cc + dd * dd)
                tv = tv + (ssum * ssum - sq)
            tot = jnp.sum(tv)
            res = jnp.where(lanei == j, tot, res)
        res = (0.5 * res + bub[pl.ds(r0, LANES)] + bib[pl.ds(r0, LANES)]
               + bgb[pl.ds(0, LANES)])
        outb[pl.ds(r0, LANES)] = res

    pltpu.sync_copy(outb, out.at[pl.ds(base, BB)])


def _pad_rows(a, n):
    return jnp.concatenate(
        [a, jnp.zeros((n - a.shape[0],) + a.shape[1:], a.dtype)], axis=0)


def kernel(user_embeddings, item_embeddings, user_feature_embeddings,
           item_feature_embeddings, context_feature_embeddings,
           user_bias, item_bias, global_bias,
           user_id, item_id, context_id,
           user_feature_mat, item_feature_mat, context_feature_mat,
           insts2userid, insts2itemid, insts2contextid):
    ue_p = _pad_rows(user_embeddings, UPAD)
    ie_p = _pad_rows(item_embeddings, UPAD)
    ufm_p = _pad_rows(user_feature_mat, UPAD)
    ifm_p = _pad_rows(item_feature_mat, UPAD)
    cfm_p = _pad_rows(context_feature_mat, CPAD)
    uf = [ufm_p[:, j] + 0 for j in range(3)]
    if_ = [ifm_p[:, j] + 0 for j in range(3)]
    cf = [cfm_p[:, j] + 0 for j in range(4)]

    npad = IPAD - NINST
    uid_p = jnp.concatenate([insts2userid, jnp.full((npad,), SENT, jnp.int32)])
    iid_p = jnp.concatenate([insts2itemid, jnp.full((npad,), SENT, jnp.int32)])
    cid_p = jnp.concatenate([insts2contextid, jnp.zeros((npad,), jnp.int32)])

    C = _pool_ctx(cf[0], cf[1], cf[2], cf[3],
                  context_feature_embeddings, context_feature_embeddings,
                  context_feature_embeddings, ie_p)
    EU = _pool_enc(ue_p, uf[0], uf[1], uf[2],
                   user_feature_embeddings, user_feature_embeddings,
                   user_feature_embeddings)
    EI = _pool_enc(ie_p, if_[0], if_[1], if_[2],
                   item_feature_embeddings, item_feature_embeddings,
                   item_feature_embeddings)

    U1 = _seg(uid_p, iid_p, cid_p, EI, C)
    I1 = _seg(iid_p, uid_p, cid_p, EU, C)

    maps = _slots(jnp.stack([user_id, item_id]))
    map_u = maps[0] + 0
    map_i = maps[1] + 0
    U2P, I2P = _seg2(uid_p, iid_p, cid_p, map_u, map_i, U1, I1, C)

    ub_flat = user_bias[:, 0] + 0
    ib_flat = item_bias[:, 0] + 0
    gb16 = jnp.broadcast_to(global_bias[0, 0], (LANES,)) + 0

    pred = _decode(user_id, item_id, context_id, cf[0], cf[1], cf[2], cf[3],
                   EU, U1, U2P, EI, I1, I2P, map_u, map_i,
                   context_feature_embeddings, ie_p,
                   ub_flat, ib_flat, gb16)
    return pred.reshape(4096, 1)


# speedup vs baseline: 1.0738x; 1.0732x over previous
import functools

import jax
import jax.numpy as jnp
from jax import lax
from jax.experimental import pallas as pl
from jax.experimental.pallas import tpu as pltpu
from jax.experimental.pallas import tpu_sc as plsc

NC, NS, LANES = 2, 16, 16
NW = NC * NS
D = 64
NUSERS = 100000
NITEMS = 100000
NCTX = 50000
NINST = 500000
UPAD = 102400
CPAD = 51200
IPAD = 524288
CH = 25600
SENT = 1 << 30

_mesh = plsc.VectorSubcoreMesh(
    core_axis_name="c", subcore_axis_name="s", num_cores=NC, num_subcores=NS)


def _wid():
    return lax.axis_index("s") * NC + lax.axis_index("c")


def _make_pool4(rows_total, chunk, n_gather, has_linear):
    iters = rows_total // (NW * chunk)
    per_w = rows_total // NW
    nsem = n_gather + (1 if has_linear else 0)
    scratch = []
    scratch += [pltpu.VMEM((chunk,), jnp.int32) for _ in range(n_gather)]
    scratch += [pltpu.VMEM((chunk, D), jnp.float32) for _ in range(n_gather)]
    if has_linear:
        scratch.append(pltpu.VMEM((chunk, D), jnp.float32))
    scratch.append(pltpu.VMEM((chunk, D), jnp.float32))
    scratch += [pltpu.SemaphoreType.DMA for _ in range(nsem)]

    scratch.append(pltpu.VMEM((chunk, D), jnp.bfloat16))

    @functools.partial(
        pl.kernel,
        out_type=(jax.ShapeDtypeStruct((rows_total, D), jnp.float32),
                  jax.ShapeDtypeStruct((rows_total, D), jnp.bfloat16)),
        mesh=_mesh,
        scratch_types=scratch,
        compiler_params=pltpu.CompilerParams(use_tc_tiling_on_sc=False, needs_layout_passes=False),
    )
    def kern(*refs):
        pos = 0
        lin = None
        if has_linear:
            lin = refs[pos]; pos += 1
        idx_hbm = refs[pos:pos + n_gather]; pos += n_gather
        tab_hbm = refs[pos:pos + n_gather]; pos += n_gather
        out = refs[pos]; pos += 1
        out_bf = refs[pos]; pos += 1
        idxb = refs[pos:pos + n_gather]; pos += n_gather
        rowb = refs[pos:pos + n_gather]; pos += n_gather
        linb = None
        if has_linear:
            linb = refs[pos]; pos += 1
        outb = refs[pos]; pos += 1
        sems = refs[pos:pos + nsem]; pos += nsem
        outb_bf = refs[pos]

        w = _wid()

        @pl.loop(0, iters)
        def _chunk(it):
            base = w * per_w + it * chunk
            cps = [pltpu.async_copy(idx_hbm[j].at[pl.ds(base, chunk)],
                                    idxb[j], sems[j])
                   for j in range(n_gather)]
            if has_linear:
                cpl = pltpu.async_copy(lin.at[pl.ds(base, chunk)], linb,
                                       sems[n_gather])
            for cp in cps:
                cp.wait()
            gps = [pltpu.async_copy(tab_hbm[j].at[idxb[j]], rowb[j], sems[j])
                   for j in range(n_gather)]
            if has_linear:
                cpl.wait()
            for gp in gps:
                gp.wait()

            @pl.loop(0, chunk)
            def _row(r):
                vs = []
                for k in range(D // LANES):
                    sl = pl.ds(k * LANES, LANES)
                    v = rowb[0][r, sl]
                    for j in range(1, n_gather):
                        v = v + rowb[j][r, sl]
                    if has_linear:
                        v = v + linb[r, sl]
                    v = v * 0.25
                    outb[r, sl] = v
                    vs.append(v)
                for h in range(D // LANES // 2):
                    outb_bf[r, pl.ds(h * 2 * LANES, 2 * LANES)] = plsc.pack(
                        vs[2 * h], vs[2 * h + 1],
                        format=plsc.PackFormat.INTERLEAVED)

            pltpu.sync_copy(outb, out.at[pl.ds(base, chunk)])
            pltpu.sync_copy(outb_bf, out_bf.at[pl.ds(base, chunk)])

    return kern


_pool_ctx = _make_pool4(CPAD, 320, 4, has_linear=False)
_pool_enc = _make_pool4(UPAD, 320, 3, has_linear=True)


IC = 2048
GR = 160
PER_TILE = IPAD // NS
ROWS_PER_TILE = CH // NS
NGMAX = (IC + GR - 1) // GR


@functools.partial(
    pl.kernel,
    out_type=jax.ShapeDtypeStruct((UPAD, D), jnp.float32),
    mesh=_mesh,
    compiler_params=pltpu.CompilerParams(use_tc_tiling_on_sc=False, needs_layout_passes=False),
    scratch_types=[
        pltpu.VMEM((IC + 176,), jnp.int32),
        pltpu.VMEM((IC + 176,), jnp.int32),
        pltpu.VMEM((IC + 176,), jnp.int32),
        pltpu.VMEM((GR,), jnp.int32),
        pltpu.VMEM((GR,), jnp.int32),
        pltpu.VMEM((GR, D), jnp.bfloat16),
        pltpu.VMEM((GR, D), jnp.bfloat16),
        pltpu.VMEM((GR, D), jnp.float32),
        pltpu.VMEM_SHARED((CH + LANES, D), jnp.float32),
        pltpu.SemaphoreType.DMA,
        pltpu.SemaphoreType.DMA,
        pltpu.SemaphoreType.DMA,
    ],
)
def _seg(dst_hbm, src_hbm, ctx_hbm, x_hbm, c_hbm, out,
         mdst, msrc, mctx, gd, gd2, rowX, rowC, rowS, acc,
         semgx, semgc, semsx):
    c = lax.axis_index("c")
    s = lax.axis_index("s")

    for p in range(2):
        chunk_id = 2 * p + c
        lo = chunk_id * CH

        @pl.loop(0, GR)
        def _z(r):
            for k in range(D // LANES):
                rowS[r, pl.ds(k * LANES, LANES)] = jnp.zeros(
                    (LANES,), jnp.float32)

        for t in range(ROWS_PER_TILE // GR):
            pltpu.sync_copy(rowS, acc.at[pl.ds(s * ROWS_PER_TILE + t * GR, GR)])

        @pl.when(s == 0)
        def _zt():
            pltpu.sync_copy(rowS.at[pl.ds(0, LANES)], acc.at[pl.ds(CH, LANES)])

        plsc.subcore_barrier()

        @pl.loop(0, PER_TILE // IC)
        def _scan(it):
            base = s * PER_TILE + it * IC
            pltpu.sync_copy(dst_hbm.at[pl.ds(base, IC)], mdst.at[pl.ds(0, IC)])
            pltpu.sync_copy(src_hbm.at[pl.ds(base, IC)], msrc.at[pl.ds(0, IC)])
            pltpu.sync_copy(ctx_hbm.at[pl.ds(base, IC)], mctx.at[pl.ds(0, IC)])

            def scan_body(g, ptr):
                sl = pl.ds(g * LANES, LANES)
                dv = mdst[sl]
                m = (dv >= lo) & (dv < lo + CH)
                cs = plsc.cumsum(m.astype(jnp.int32))
                pos = ptr + cs - 1
                plsc.store_scatter(mdst, [pos], dv - lo, mask=m)
                plsc.store_scatter(msrc, [pos], msrc[sl], mask=m)
                plsc.store_scatter(mctx, [pos], mctx[sl], mask=m)
                return ptr + plsc.all_reduce_population_count(m)[0]

            ptr = lax.fori_loop(0, IC // LANES, scan_body, jnp.int32(0),
                                unroll=4)

            tvec = jnp.int32(CH) + lax.iota(jnp.int32, LANES)
            zv = jnp.zeros((LANES,), jnp.int32)
            for k in range(GR // LANES):
                mdst[pl.ds(ptr + k * LANES, LANES)] = tvec
                msrc[pl.ds(ptr + k * LANES, LANES)] = zv
                mctx[pl.ds(ptr + k * LANES, LANES)] = zv

            ng = (ptr + (GR - 1)) // GR
            descs = {}
            for k in range(NGMAX):
                @pl.when(k < ng)
                def _gran(k=k):
                    gdk = gd if k % 2 == 0 else gd2
                    gX = pltpu.async_copy(
                        x_hbm.at[msrc.at[pl.ds(k * GR, GR)]], rowX, semgx)
                    gC = pltpu.async_copy(
                        c_hbm.at[mctx.at[pl.ds(k * GR, GR)]], rowC, semgc)
                    for v in range(GR // LANES):
                        gdk[pl.ds(v * LANES, LANES)] = (
                            mdst[pl.ds(k * GR + v * LANES, LANES)])
                    gX.wait()
                    gC.wait()
                    if k > 0:
                        descs[k - 1].wait()

                    @pl.loop(0, GR)
                    def _add(r):
                        for h in range(D // LANES // 2):
                            sl2 = pl.ds(h * 2 * LANES, 2 * LANES)
                            xa, xb = plsc.unpack(
                                rowX[r, sl2], format=plsc.PackFormat.INTERLEAVED)
                            ca, cb = plsc.unpack(
                                rowC[r, sl2], format=plsc.PackFormat.INTERLEAVED)
                            rowS[r, pl.ds(2 * h * LANES, LANES)] = xa + ca
                            rowS[r, pl.ds((2 * h + 1) * LANES, LANES)] = xb + cb

                    descs[k] = pltpu.async_copy(rowS, acc.at[gdk], semsx,
                                                add=True)

                    @pl.when(k == ng - 1)
                    def _last():
                        descs[k].wait()

        plsc.subcore_barrier()

        for t in range(ROWS_PER_TILE // GR):
            r0 = s * ROWS_PER_TILE + t * GR
            pltpu.sync_copy(acc.at[pl.ds(r0, GR)], out.at[pl.ds(lo + r0, GR)])

        plsc.subcore_barrier()


MS = 103424
MZC = 6464
BPT = 4096 // NS


@functools.partial(
    pl.kernel,
    out_type=jax.ShapeDtypeStruct((2, MS), jnp.int32),
    mesh=_mesh,
    compiler_params=pltpu.CompilerParams(use_tc_tiling_on_sc=False, needs_layout_passes=False),
    scratch_types=[
        pltpu.VMEM((MZC // 4,), jnp.int32),
        pltpu.VMEM((BPT,), jnp.int32),
        pltpu.VMEM((BPT,), jnp.int32),
    ],
)
def _slots(ids2_hbm, maps, zb, idb, valb):
    c = lax.axis_index("c")
    s = lax.axis_index("s")

    @pl.loop(0, MZC // 4 // LANES)
    def _zf(g):
        zb[pl.ds(g * LANES, LANES)] = jnp.zeros((LANES,), jnp.int32)

    @pl.loop(0, BPT // LANES)
    def _vf(g):
        valb[pl.ds(g * LANES, LANES)] = (
            s * BPT + g * LANES + 1 + lax.iota(jnp.int32, LANES))

    for t in range(4):
        pltpu.sync_copy(zb, maps.at[c].at[pl.ds(s * MZC + t * (MZC // 4),
                                                MZC // 4)])
    plsc.subcore_barrier()
    pltpu.sync_copy(ids2_hbm.at[c].at[pl.ds(s * BPT, BPT)], idb)
    pltpu.sync_copy(valb, maps.at[c].at[idb])


NSLOT = 4096
IC2 = 2048
ZB = 64
PT2 = IPAD // NW


@functools.partial(
    pl.kernel,
    out_type=(jax.ShapeDtypeStruct((2 * NSLOT, D), jnp.float32),
              jax.ShapeDtypeStruct((2 * NSLOT, D), jnp.float32)),
    mesh=_mesh,
    compiler_params=pltpu.CompilerParams(use_tc_tiling_on_sc=False, needs_layout_passes=False),
    scratch_types=[
        pltpu.VMEM((IC2,), jnp.int32),
        pltpu.VMEM((IC2,), jnp.int32),
        pltpu.VMEM((IC2,), jnp.int32),
        pltpu.VMEM((IC2,), jnp.int32),
        pltpu.VMEM((IC2,), jnp.int32),
        pltpu.VMEM((IC2 + 176,), jnp.int32),
        pltpu.VMEM((IC2 + 176,), jnp.int32),
        pltpu.VMEM((IC2 + 176,), jnp.int32),
        pltpu.VMEM((IC2 + 176,), jnp.int32),
        pltpu.VMEM((IC2 + 176,), jnp.int32),
        pltpu.VMEM((IC2 + 176,), jnp.int32),
        pltpu.VMEM((GR,), jnp.int32),
        pltpu.VMEM((GR,), jnp.int32),
        pltpu.VMEM((GR, D), jnp.float32),
        pltpu.VMEM((GR, D), jnp.float32),
        pltpu.VMEM((ZB, D), jnp.float32),
        pltpu.VMEM_SHARED((NSLOT + LANES, D), jnp.float32),
        pltpu.VMEM_SHARED((NSLOT + LANES, D), jnp.float32),
        pltpu.SemaphoreType.DMA,
        pltpu.SemaphoreType.DMA,
        pltpu.SemaphoreType.DMA,
        pltpu.SemaphoreType.DMA,
    ],
)
def _seg2(uid_hbm, iid_hbm, cid_hbm, mapu_hbm, mapi_hbm,
          u1_hbm, i1_hbm, c_hbm, out_u, out_i,
          ubuf, ibuf, cbuf, su, si,
          mu_dst, mu_src, mu_ctx, mi_dst, mi_src, mi_ctx,
          gd, gd2, rowX, rowC, zbuf, acc_u, acc_i,
          semgx, semgc, semsx, semsc):
    c = lax.axis_index("c")
    s = lax.axis_index("s")

    @pl.loop(0, ZB)
    def _z(r):
        for k in range(D // LANES):
            zbuf[r, pl.ds(k * LANES, LANES)] = jnp.zeros((LANES,), jnp.float32)

    rpt = NSLOT // NS
    for acc in (acc_u, acc_i):
        for t in range(rpt // ZB):
            pltpu.sync_copy(zbuf, acc.at[pl.ds(s * rpt + t * ZB, ZB)])

        @pl.when(s == 0)
        def _zt():
            pltpu.sync_copy(zbuf.at[pl.ds(0, LANES)],
                            acc.at[pl.ds(NSLOT, LANES)])

    plsc.subcore_barrier()

    @pl.loop(0, PT2 // IC2)
    def _scan(it):
        base = (c * NS + s) * PT2 + it * IC2
        pltpu.sync_copy(uid_hbm.at[pl.ds(base, IC2)], ubuf)
        pltpu.sync_copy(iid_hbm.at[pl.ds(base, IC2)], ibuf)
        pltpu.sync_copy(cid_hbm.at[pl.ds(base, IC2)], cbuf)
        pltpu.sync_copy(mapu_hbm.at[ubuf], su)
        pltpu.sync_copy(mapi_hbm.at[ibuf], si)

        def scan_body(g, carry):
            pu, pi = carry
            sl = pl.ds(g * LANES, LANES)
            sv = su[sl]
            mu = sv > 0
            csu = plsc.cumsum(mu.astype(jnp.int32))
            posu = pu + csu - 1
            plsc.store_scatter(mu_dst, [posu], sv - 1, mask=mu)
            plsc.store_scatter(mu_src, [posu], ibuf[sl], mask=mu)
            plsc.store_scatter(mu_ctx, [posu], cbuf[sl], mask=mu)
            tv = si[sl]
            mi = tv > 0
            csi = plsc.cumsum(mi.astype(jnp.int32))
            posi = pi + csi - 1
            plsc.store_scatter(mi_dst, [posi], tv - 1, mask=mi)
            plsc.store_scatter(mi_src, [posi], ubuf[sl], mask=mi)
            plsc.store_scatter(mi_ctx, [posi], cbuf[sl], mask=mi)
            return (pu + plsc.all_reduce_population_count(mu)[0],
                    pi + plsc.all_reduce_population_count(mi)[0])

        pu, pi = lax.fori_loop(0, IC2 // LANES, scan_body,
                               (jnp.int32(0), jnp.int32(0)), unroll=4)

        tvec = jnp.int32(NSLOT) + lax.iota(jnp.int32, LANES)
        zv = jnp.zeros((LANES,), jnp.int32)
        for k in range(GR // LANES):
            mu_dst[pl.ds(pu + k * LANES, LANES)] = tvec
            mu_src[pl.ds(pu + k * LANES, LANES)] = zv
            mu_ctx[pl.ds(pu + k * LANES, LANES)] = zv
            mi_dst[pl.ds(pi + k * LANES, LANES)] = tvec
            mi_src[pl.ds(pi + k * LANES, LANES)] = zv
            mi_ctx[pl.ds(pi + k * LANES, LANES)] = zv

        for (md, msrc_, mc, xh, accr, ptr) in (
                (mu_dst, mu_src, mu_ctx, i1_hbm, acc_u, pu),
                (mi_dst, mi_src, mi_ctx, u1_hbm, acc_i, pi)):
            ng = (ptr + (GR - 1)) // GR
            descs = {}
            for k in range(NGMAX):
                @pl.when(k < ng)
                def _gran(k=k, md=md, msrc_=msrc_, mc=mc, xh=xh, accr=accr):
                    gdk = gd if k % 2 == 0 else gd2
                    gX = pltpu.async_copy(
                        xh.at[msrc_.at[pl.ds(k * GR, GR)]], rowX, semgx)
                    for v in range(GR // LANES):
                        gdk[pl.ds(v * LANES, LANES)] = (
                            md[pl.ds(k * GR + v * LANES, LANES)])
                    if k > 0:
                        descs[k - 1].wait()
                    gC = pltpu.async_copy(
                        c_hbm.at[mc.at[pl.ds(k * GR, GR)]], rowC, semgc)
                    gX.wait()
                    gC.wait()

                    @pl.loop(0, GR)
                    def _add(r):
                        for v in range(D // LANES):
                            sl = pl.ds(v * LANES, LANES)
                            rowC[r, sl] = rowX[r, sl] + rowC[r, sl]

                    descs[k] = pltpu.async_copy(rowC, accr.at[gdk], semsx,
                                                add=True)

                    @pl.when(k == ng - 1)
                    def _last():
                        descs[k].wait()

    plsc.subcore_barrier()

    rpt = NSLOT // NS
    for (accr, outr) in ((acc_u, out_u), (acc_i, out_i)):
        for t in range(rpt // ZB):
            r0 = s * rpt + t * ZB
            pltpu.sync_copy(accr.at[pl.ds(r0, ZB)],
                            outr.at[pl.ds(c * NSLOT + r0, ZB)])


BB = 4096 // NW


@functools.partial(
    pl.kernel,
    out_type=jax.ShapeDtypeStruct((4096,), jnp.float32),
    mesh=_mesh,
    compiler_params=pltpu.CompilerParams(use_tc_tiling_on_sc=False, needs_layout_passes=False),
    scratch_types=(
        [pltpu.VMEM((BB,), jnp.int32) for _ in range(7)]
        + [pltpu.VMEM((BB,), jnp.int32) for _ in range(6)]
        + [pltpu.VMEM((BB, D), jnp.float32) for _ in range(12)]
        + [pltpu.VMEM((BB,), jnp.float32) for _ in range(2)]
        + [pltpu.VMEM((LANES,), jnp.float32), pltpu.VMEM((BB,), jnp.float32)]
    ),
)
def _decode(uid_hbm, iid_hbm, cid_hbm, cf0, cf1, cf2, cf3,
            eu_hbm, u1_hbm, u2p_hbm, ei_hbm, i1_hbm, i2p_hbm,
            mapu_hbm, mapi_hbm,
            cfe_hbm, ie_hbm, ub_hbm, ib_hbm, gb_hbm, out,
            ub, ib, cb, cm0, cm1, cm2, cm3,
            su, si, ju0, ju1, ji0, ji1,
            bEu, bU1, bU2a, bU2b, bEi, bI1, bI2a, bI2b, r2, r3, r4, r5,
            bub, bib, bgb, outb):
    w = _wid()
    base = w * BB
    pltpu.sync_copy(uid_hbm.at[pl.ds(base, BB)], ub)
    pltpu.sync_copy(iid_hbm.at[pl.ds(base, BB)], ib)
    pltpu.sync_copy(cid_hbm.at[pl.ds(base, BB)], cb)
    pltpu.sync_copy(cf0.at[cb], cm0)
    pltpu.sync_copy(cf1.at[cb], cm1)
    pltpu.sync_copy(cf2.at[cb], cm2)
    pltpu.sync_copy(cf3.at[cb], cm3)
    pltpu.sync_copy(mapu_hbm.at[ub], su)
    pltpu.sync_copy(mapi_hbm.at[ib], si)

    @pl.loop(0, BB // LANES)
    def _ji(g):
        sl = pl.ds(g * LANES, LANES)
        vu = su[sl] - 1
        ju0[sl] = vu
        ju1[sl] = vu + NSLOT
        vi = si[sl] - 1
        ji0[sl] = vi
        ji1[sl] = vi + NSLOT

    pltpu.sync_copy(eu_hbm.at[ub], bEu)
    pltpu.sync_copy(u1_hbm.at[ub], bU1)
    pltpu.sync_copy(u2p_hbm.at[ju0], bU2a)
    pltpu.sync_copy(u2p_hbm.at[ju1], bU2b)
    pltpu.sync_copy(ei_hbm.at[ib], bEi)
    pltpu.sync_copy(i1_hbm.at[ib], bI1)
    pltpu.sync_copy(i2p_hbm.at[ji0], bI2a)
    pltpu.sync_copy(i2p_hbm.at[ji1], bI2b)
    pltpu.sync_copy(cfe_hbm.at[cm0], r2)
    pltpu.sync_copy(cfe_hbm.at[cm1], r3)
    pltpu.sync_copy(cfe_hbm.at[cm2], r4)
    pltpu.sync_copy(ie_hbm.at[cm3], r5)
    pltpu.sync_copy(ub_hbm.at[ub], bub)
    pltpu.sync_copy(ib_hbm.at[ib], bib)
    pltpu.sync_copy(gb_hbm, bgb)

    @pl.loop(0, BB // LANES)
    def _grp(g):
        r0 = g * LANES
        lanei = lax.iota(jnp.int32, LANES)
        res = jnp.zeros((LANES,), jnp.float32)
        for j in range(LANES):
            r = r0 + j
            tv = jnp.zeros((LANES,), jnp.float32)
            for k in range(D // LANES):
                sl = pl.ds(k * LANES, LANES)
                fu = (bEu[r, sl] + 0.5 * bU1[r, sl]
                      + 0.25 * (bU2a[r, sl] + bU2b[r, sl]))
                fi = (bEi[r, sl] + 0.5 * bI1[r, sl]
                      + 0.25 * (bI2a[r, sl] + bI2b[r, sl]))
                a = r2[r, sl]
                b = r3[r, sl]
                cc = r4[r, sl]
                dd = r5[r, sl]
                ssum = fu + fi + a + b + cc + dd
                sq = (fu * fu + fi * fi + a * a + b * b + cc * cc + dd * dd)
                tv = tv + (ssum * ssum - sq)
            tot = jnp.sum(tv)
            res = jnp.where(lanei == j, tot, res)
        res = (0.5 * res + bub[pl.ds(r0, LANES)] + bib[pl.ds(r0, LANES)]
               + bgb[pl.ds(0, LANES)])
        outb[pl.ds(r0, LANES)] = res

    pltpu.sync_copy(outb, out.at[pl.ds(base, BB)])


def _pad_rows(a, n):
    return jnp.concatenate(
        [a, jnp.zeros((n - a.shape[0],) + a.shape[1:], a.dtype)], axis=0)


def kernel(user_embeddings, item_embeddings, user_feature_embeddings,
           item_feature_embeddings, context_feature_embeddings,
           user_bias, item_bias, global_bias,
           user_id, item_id, context_id,
           user_feature_mat, item_feature_mat, context_feature_mat,
           insts2userid, insts2itemid, insts2contextid):
    ue_p = _pad_rows(user_embeddings, UPAD)
    ie_p = _pad_rows(item_embeddings, UPAD)
    ufm_p = _pad_rows(user_feature_mat, UPAD)
    ifm_p = _pad_rows(item_feature_mat, UPAD)
    cfm_p = _pad_rows(context_feature_mat, CPAD)
    uf = [ufm_p[:, j] + 0 for j in range(3)]
    if_ = [ifm_p[:, j] + 0 for j in range(3)]
    cf = [cfm_p[:, j] + 0 for j in range(4)]

    npad = IPAD - NINST
    uid_p = jnp.concatenate([insts2userid, jnp.full((npad,), SENT, jnp.int32)])
    iid_p = jnp.concatenate([insts2itemid, jnp.full((npad,), SENT, jnp.int32)])
    cid_p = jnp.concatenate([insts2contextid, jnp.zeros((npad,), jnp.int32)])

    C, C_bf = _pool_ctx(cf[0], cf[1], cf[2], cf[3],
                        context_feature_embeddings, context_feature_embeddings,
                        context_feature_embeddings, ie_p)
    EU, EU_bf = _pool_enc(ue_p, uf[0], uf[1], uf[2],
                          user_feature_embeddings, user_feature_embeddings,
                          user_feature_embeddings)
    EI, EI_bf = _pool_enc(ie_p, if_[0], if_[1], if_[2],
                          item_feature_embeddings, item_feature_embeddings,
                          item_feature_embeddings)

    U1 = _seg(uid_p, iid_p, cid_p, EI_bf, C_bf)
    I1 = _seg(iid_p, uid_p, cid_p, EU_bf, C_bf)

    maps = _slots(jnp.stack([user_id, item_id]))
    map_u = maps[0] + 0
    map_i = maps[1] + 0
    U2P, I2P = _seg2(uid_p, iid_p, cid_p, map_u, map_i, U1, I1, C)

    ub_flat = user_bias[:, 0] + 0
    ib_flat = item_bias[:, 0] + 0
    gb16 = jnp.broadcast_to(global_bias[0, 0], (LANES,)) + 0

    pred = _decode(user_id, item_id, context_id, cf[0], cf[1], cf[2], cf[3],
                   EU, U1, U2P, EI, I1, I2P, map_u, map_i,
                   context_feature_embeddings, ie_p,
                   ub_flat, ib_flat, gb16)
    return pred.reshape(4096, 1)


# speedup vs baseline: 1.2746x; 1.1870x over previous
import functools

import jax
import jax.numpy as jnp
from jax import lax
from jax.experimental import pallas as pl
from jax.experimental.pallas import tpu as pltpu
from jax.experimental.pallas import tpu_sc as plsc

NC, NS, LANES = 2, 16, 16
NW = NC * NS
D = 64
NUSERS = 100000
NITEMS = 100000
NCTX = 50000
NINST = 500000
UPAD = 102400
CPAD = 51200
IPAD = 524288
CH = 25600
SENT = 1 << 30

_mesh = plsc.VectorSubcoreMesh(
    core_axis_name="c", subcore_axis_name="s", num_cores=NC, num_subcores=NS)


def _wid():
    return lax.axis_index("s") * NC + lax.axis_index("c")


def _make_pool4(rows_total, chunk, n_gather, has_linear, table_rows, n_spmem):
    iters = rows_total // (NW * chunk)
    per_w = rows_total // NW
    nsem = n_gather + (1 if has_linear else 0)
    scratch = []
    scratch += [pltpu.VMEM((chunk,), jnp.int32) for _ in range(n_gather)]
    scratch += [pltpu.VMEM((chunk, D), jnp.float32) for _ in range(n_gather)]
    if has_linear:
        scratch.append(pltpu.VMEM((chunk, D), jnp.float32))
    scratch.append(pltpu.VMEM((chunk, D), jnp.float32))
    scratch += [pltpu.SemaphoreType.DMA for _ in range(nsem)]

    scratch.append(pltpu.VMEM((chunk, D), jnp.bfloat16))
    scratch.append(pltpu.VMEM_SHARED((table_rows, D), jnp.float32))

    @functools.partial(
        pl.kernel,
        out_type=(jax.ShapeDtypeStruct((rows_total, D), jnp.float32),
                  jax.ShapeDtypeStruct((rows_total, D), jnp.bfloat16)),
        mesh=_mesh,
        scratch_types=scratch,
        compiler_params=pltpu.CompilerParams(use_tc_tiling_on_sc=False, needs_layout_passes=False),
    )
    def kern(*refs):
        pos = 0
        lin = None
        if has_linear:
            lin = refs[pos]; pos += 1
        idx_hbm = refs[pos:pos + n_gather]; pos += n_gather
        tab_hbm = refs[pos:pos + n_gather]; pos += n_gather
        out = refs[pos]; pos += 1
        out_bf = refs[pos]; pos += 1
        idxb = refs[pos:pos + n_gather]; pos += n_gather
        rowb = refs[pos:pos + n_gather]; pos += n_gather
        linb = None
        if has_linear:
            linb = refs[pos]; pos += 1
        outb = refs[pos]; pos += 1
        sems = refs[pos:pos + nsem]; pos += nsem
        outb_bf = refs[pos]; pos += 1
        spmem_tab = refs[pos]

        w = _wid()
        s = lax.axis_index("s")

        @pl.when(s == 0)
        def _stage():
            pltpu.sync_copy(tab_hbm[0], spmem_tab)

        plsc.subcore_barrier()

        @pl.loop(0, iters)
        def _chunk(it):
            base = w * per_w + it * chunk
            cps = [pltpu.async_copy(idx_hbm[j].at[pl.ds(base, chunk)],
                                    idxb[j], sems[j])
                   for j in range(n_gather)]
            if has_linear:
                cpl = pltpu.async_copy(lin.at[pl.ds(base, chunk)], linb,
                                       sems[n_gather])
            for cp in cps:
                cp.wait()
            gps = [pltpu.async_copy(
                       (spmem_tab if j < n_spmem else tab_hbm[j]).at[idxb[j]],
                       rowb[j], sems[j])
                   for j in range(n_gather)]
            if has_linear:
                cpl.wait()
            for gp in gps:
                gp.wait()

            @pl.loop(0, chunk)
            def _row(r):
                vs = []
                for k in range(D // LANES):
                    sl = pl.ds(k * LANES, LANES)
                    v = rowb[0][r, sl]
                    for j in range(1, n_gather):
                        v = v + rowb[j][r, sl]
                    if has_linear:
                        v = v + linb[r, sl]
                    v = v * 0.25
                    outb[r, sl] = v
                    vs.append(v)
                for h in range(D // LANES // 2):
                    outb_bf[r, pl.ds(h * 2 * LANES, 2 * LANES)] = plsc.pack(
                        vs[2 * h], vs[2 * h + 1],
                        format=plsc.PackFormat.INTERLEAVED)

            pltpu.sync_copy(outb, out.at[pl.ds(base, chunk)])
            pltpu.sync_copy(outb_bf, out_bf.at[pl.ds(base, chunk)])

    return kern


_pool_ctx = _make_pool4(CPAD, 320, 4, has_linear=False,
                        table_rows=2000, n_spmem=3)
_pool_enc_u = _make_pool4(UPAD, 320, 3, has_linear=True,
                          table_rows=1000, n_spmem=3)
_pool_enc_i = _make_pool4(UPAD, 320, 3, has_linear=True,
                          table_rows=2000, n_spmem=3)


IC = 2048
GR = 160
PER_TILE = IPAD // NS
ROWS_PER_TILE = CH // NS
NGMAX = (IC + GR - 1) // GR


@functools.partial(
    pl.kernel,
    out_type=jax.ShapeDtypeStruct((UPAD, D), jnp.float32),
    mesh=_mesh,
    compiler_params=pltpu.CompilerParams(use_tc_tiling_on_sc=False, needs_layout_passes=False),
    scratch_types=[
        pltpu.VMEM((IC + 176,), jnp.int32),
        pltpu.VMEM((IC + 176,), jnp.int32),
        pltpu.VMEM((IC + 176,), jnp.int32),
        pltpu.VMEM((GR,), jnp.int32),
        pltpu.VMEM((GR,), jnp.int32),
        pltpu.VMEM((GR, D), jnp.bfloat16),
        pltpu.VMEM((GR, D), jnp.bfloat16),
        pltpu.VMEM((GR, D), jnp.float32),
        pltpu.VMEM_SHARED((CH + LANES, D), jnp.float32),
        pltpu.SemaphoreType.DMA,
        pltpu.SemaphoreType.DMA,
        pltpu.SemaphoreType.DMA,
    ],
)
def _seg(dst_hbm, src_hbm, ctx_hbm, x_hbm, c_hbm, out,
         mdst, msrc, mctx, gd, gd2, rowX, rowC, rowS, acc,
         semgx, semgc, semsx):
    c = lax.axis_index("c")
    s = lax.axis_index("s")

    for p in range(2):
        chunk_id = 2 * p + c
        lo = chunk_id * CH

        @pl.loop(0, GR)
        def _z(r):
            for k in range(D // LANES):
                rowS[r, pl.ds(k * LANES, LANES)] = jnp.zeros(
                    (LANES,), jnp.float32)

        for t in range(ROWS_PER_TILE // GR):
            pltpu.sync_copy(rowS, acc.at[pl.ds(s * ROWS_PER_TILE + t * GR, GR)])

        @pl.when(s == 0)
        def _zt():
            pltpu.sync_copy(rowS.at[pl.ds(0, LANES)], acc.at[pl.ds(CH, LANES)])

        plsc.subcore_barrier()

        @pl.loop(0, PER_TILE // IC)
        def _scan(it):
            base = s * PER_TILE + it * IC
            pltpu.sync_copy(dst_hbm.at[pl.ds(base, IC)], mdst.at[pl.ds(0, IC)])
            pltpu.sync_copy(src_hbm.at[pl.ds(base, IC)], msrc.at[pl.ds(0, IC)])
            pltpu.sync_copy(ctx_hbm.at[pl.ds(base, IC)], mctx.at[pl.ds(0, IC)])

            def scan_body(g, ptr):
                sl = pl.ds(g * LANES, LANES)
                dv = mdst[sl]
                m = (dv >= lo) & (dv < lo + CH)
                cs = plsc.cumsum(m.astype(jnp.int32))
                pos = ptr + cs - 1
                plsc.store_scatter(mdst, [pos], dv - lo, mask=m)
                plsc.store_scatter(msrc, [pos], msrc[sl], mask=m)
                plsc.store_scatter(mctx, [pos], mctx[sl], mask=m)
                return ptr + plsc.all_reduce_population_count(m)[0]

            ptr = lax.fori_loop(0, IC // LANES, scan_body, jnp.int32(0),
                                unroll=4)

            tvec = jnp.int32(CH) + lax.iota(jnp.int32, LANES)
            zv = jnp.zeros((LANES,), jnp.int32)
            for k in range(GR // LANES):
                mdst[pl.ds(ptr + k * LANES, LANES)] = tvec
                msrc[pl.ds(ptr + k * LANES, LANES)] = zv
                mctx[pl.ds(ptr + k * LANES, LANES)] = zv

            ng = (ptr + (GR - 1)) // GR
            descs = {}
            for k in range(NGMAX):
                @pl.when(k < ng)
                def _gran(k=k):
                    gdk = gd if k % 2 == 0 else gd2
                    gX = pltpu.async_copy(
                        x_hbm.at[msrc.at[pl.ds(k * GR, GR)]], rowX, semgx)
                    gC = pltpu.async_copy(
                        c_hbm.at[mctx.at[pl.ds(k * GR, GR)]], rowC, semgc)
                    for v in range(GR // LANES):
                        gdk[pl.ds(v * LANES, LANES)] = (
                            mdst[pl.ds(k * GR + v * LANES, LANES)])
                    gX.wait()
                    gC.wait()
                    if k > 0:
                        descs[k - 1].wait()

                    @pl.loop(0, GR)
                    def _add(r):
                        for h in range(D // LANES // 2):
                            sl2 = pl.ds(h * 2 * LANES, 2 * LANES)
                            xa, xb = plsc.unpack(
                                rowX[r, sl2], format=plsc.PackFormat.INTERLEAVED)
                            ca, cb = plsc.unpack(
                                rowC[r, sl2], format=plsc.PackFormat.INTERLEAVED)
                            rowS[r, pl.ds(2 * h * LANES, LANES)] = xa + ca
                            rowS[r, pl.ds((2 * h + 1) * LANES, LANES)] = xb + cb

                    descs[k] = pltpu.async_copy(rowS, acc.at[gdk], semsx,
                                                add=True)

                    @pl.when(k == ng - 1)
                    def _last():
                        descs[k].wait()

        plsc.subcore_barrier()

        for t in range(ROWS_PER_TILE // GR):
            r0 = s * ROWS_PER_TILE + t * GR
            pltpu.sync_copy(acc.at[pl.ds(r0, GR)], out.at[pl.ds(lo + r0, GR)])

        plsc.subcore_barrier()


MS = 103424
MZC = 6464
BPT = 4096 // NS


@functools.partial(
    pl.kernel,
    out_type=jax.ShapeDtypeStruct((2, MS), jnp.int32),
    mesh=_mesh,
    compiler_params=pltpu.CompilerParams(use_tc_tiling_on_sc=False, needs_layout_passes=False),
    scratch_types=[
        pltpu.VMEM((MZC // 4,), jnp.int32),
        pltpu.VMEM((BPT,), jnp.int32),
        pltpu.VMEM((BPT,), jnp.int32),
    ],
)
def _slots(ids2_hbm, maps, zb, idb, valb):
    c = lax.axis_index("c")
    s = lax.axis_index("s")

    @pl.loop(0, MZC // 4 // LANES)
    def _zf(g):
        zb[pl.ds(g * LANES, LANES)] = jnp.zeros((LANES,), jnp.int32)

    @pl.loop(0, BPT // LANES)
    def _vf(g):
        valb[pl.ds(g * LANES, LANES)] = (
            s * BPT + g * LANES + 1 + lax.iota(jnp.int32, LANES))

    for t in range(4):
        pltpu.sync_copy(zb, maps.at[c].at[pl.ds(s * MZC + t * (MZC // 4),
                                                MZC // 4)])
    plsc.subcore_barrier()
    pltpu.sync_copy(ids2_hbm.at[c].at[pl.ds(s * BPT, BPT)], idb)
    pltpu.sync_copy(valb, maps.at[c].at[idb])


NSLOT = 4096
IC2 = 2048
ZB = 64
PT2 = IPAD // NW


@functools.partial(
    pl.kernel,
    out_type=(jax.ShapeDtypeStruct((2 * NSLOT, D), jnp.float32),
              jax.ShapeDtypeStruct((2 * NSLOT, D), jnp.float32)),
    mesh=_mesh,
    compiler_params=pltpu.CompilerParams(use_tc_tiling_on_sc=False, needs_layout_passes=False),
    scratch_types=[
        pltpu.VMEM((IC2,), jnp.int32),
        pltpu.VMEM((IC2,), jnp.int32),
        pltpu.VMEM((IC2,), jnp.int32),
        pltpu.VMEM((IC2,), jnp.int32),
        pltpu.VMEM((IC2,), jnp.int32),
        pltpu.VMEM((IC2 + 176,), jnp.int32),
        pltpu.VMEM((IC2 + 176,), jnp.int32),
        pltpu.VMEM((IC2 + 176,), jnp.int32),
        pltpu.VMEM((IC2 + 176,), jnp.int32),
        pltpu.VMEM((IC2 + 176,), jnp.int32),
        pltpu.VMEM((IC2 + 176,), jnp.int32),
        pltpu.VMEM((GR,), jnp.int32),
        pltpu.VMEM((GR,), jnp.int32),
        pltpu.VMEM((GR, D), jnp.float32),
        pltpu.VMEM((GR, D), jnp.float32),
        pltpu.VMEM((ZB, D), jnp.float32),
        pltpu.VMEM_SHARED((NSLOT + LANES, D), jnp.float32),
        pltpu.VMEM_SHARED((NSLOT + LANES, D), jnp.float32),
        pltpu.VMEM_SHARED((MS,), jnp.int32),
        pltpu.VMEM_SHARED((MS,), jnp.int32),
        pltpu.SemaphoreType.DMA,
        pltpu.SemaphoreType.DMA,
        pltpu.SemaphoreType.DMA,
        pltpu.SemaphoreType.DMA,
    ],
)
def _seg2(uid_hbm, iid_hbm, cid_hbm, mapu_hbm, mapi_hbm,
          u1_hbm, i1_hbm, c_hbm, out_u, out_i,
          ubuf, ibuf, cbuf, su, si,
          mu_dst, mu_src, mu_ctx, mi_dst, mi_src, mi_ctx,
          gd, gd2, rowX, rowC, zbuf, acc_u, acc_i, smap_u, smap_i,
          semgx, semgc, semsx, semsc):
    c = lax.axis_index("c")
    s = lax.axis_index("s")

    @pl.when(s == 0)
    def _stage_maps():
        pltpu.sync_copy(mapu_hbm, smap_u)
        pltpu.sync_copy(mapi_hbm, smap_i)

    @pl.loop(0, ZB)
    def _z(r):
        for k in range(D // LANES):
            zbuf[r, pl.ds(k * LANES, LANES)] = jnp.zeros((LANES,), jnp.float32)

    rpt = NSLOT // NS
    for acc in (acc_u, acc_i):
        for t in range(rpt // ZB):
            pltpu.sync_copy(zbuf, acc.at[pl.ds(s * rpt + t * ZB, ZB)])

        @pl.when(s == 0)
        def _zt():
            pltpu.sync_copy(zbuf.at[pl.ds(0, LANES)],
                            acc.at[pl.ds(NSLOT, LANES)])

    plsc.subcore_barrier()

    @pl.loop(0, PT2 // IC2)
    def _scan(it):
        base = (c * NS + s) * PT2 + it * IC2
        pltpu.sync_copy(uid_hbm.at[pl.ds(base, IC2)], ubuf)
        pltpu.sync_copy(iid_hbm.at[pl.ds(base, IC2)], ibuf)
        pltpu.sync_copy(cid_hbm.at[pl.ds(base, IC2)], cbuf)
        pltpu.sync_copy(smap_u.at[ubuf], su)
        pltpu.sync_copy(smap_i.at[ibuf], si)

        def scan_body(g, carry):
            pu, pi = carry
            sl = pl.ds(g * LANES, LANES)
            sv = su[sl]
            mu = sv > 0
            csu = plsc.cumsum(mu.astype(jnp.int32))
            posu = pu + csu - 1
            plsc.store_scatter(mu_dst, [posu], sv - 1, mask=mu)
            plsc.store_scatter(mu_src, [posu], ibuf[sl], mask=mu)
            plsc.store_scatter(mu_ctx, [posu], cbuf[sl], mask=mu)
            tv = si[sl]
            mi = tv > 0
            csi = plsc.cumsum(mi.astype(jnp.int32))
            posi = pi + csi - 1
            plsc.store_scatter(mi_dst, [posi], tv - 1, mask=mi)
            plsc.store_scatter(mi_src, [posi], ubuf[sl], mask=mi)
            plsc.store_scatter(mi_ctx, [posi], cbuf[sl], mask=mi)
            return (pu + plsc.all_reduce_population_count(mu)[0],
                    pi + plsc.all_reduce_population_count(mi)[0])

        pu, pi = lax.fori_loop(0, IC2 // LANES, scan_body,
                               (jnp.int32(0), jnp.int32(0)), unroll=4)

        tvec = jnp.int32(NSLOT) + lax.iota(jnp.int32, LANES)
        zv = jnp.zeros((LANES,), jnp.int32)
        for k in range(GR // LANES):
            mu_dst[pl.ds(pu + k * LANES, LANES)] = tvec
            mu_src[pl.ds(pu + k * LANES, LANES)] = zv
            mu_ctx[pl.ds(pu + k * LANES, LANES)] = zv
            mi_dst[pl.ds(pi + k * LANES, LANES)] = tvec
            mi_src[pl.ds(pi + k * LANES, LANES)] = zv
            mi_ctx[pl.ds(pi + k * LANES, LANES)] = zv

        for (md, msrc_, mc, xh, accr, ptr) in (
                (mu_dst, mu_src, mu_ctx, i1_hbm, acc_u, pu),
                (mi_dst, mi_src, mi_ctx, u1_hbm, acc_i, pi)):
            ng = (ptr + (GR - 1)) // GR
            descs = {}
            for k in range(NGMAX):
                @pl.when(k < ng)
                def _gran(k=k, md=md, msrc_=msrc_, mc=mc, xh=xh, accr=accr):
                    gdk = gd if k % 2 == 0 else gd2
                    gX = pltpu.async_copy(
                        xh.at[msrc_.at[pl.ds(k * GR, GR)]], rowX, semgx)
                    for v in range(GR // LANES):
                        gdk[pl.ds(v * LANES, LANES)] = (
                            md[pl.ds(k * GR + v * LANES, LANES)])
                    if k > 0:
                        descs[k - 1].wait()
                    gC = pltpu.async_copy(
                        c_hbm.at[mc.at[pl.ds(k * GR, GR)]], rowC, semgc)
                    gX.wait()
                    gC.wait()

                    @pl.loop(0, GR)
                    def _add(r):
                        for v in range(D // LANES):
                            sl = pl.ds(v * LANES, LANES)
                            rowC[r, sl] = rowX[r, sl] + rowC[r, sl]

                    descs[k] = pltpu.async_copy(rowC, accr.at[gdk], semsx,
                                                add=True)

                    @pl.when(k == ng - 1)
                    def _last():
                        descs[k].wait()

    plsc.subcore_barrier()

    rpt = NSLOT // NS
    for (accr, outr) in ((acc_u, out_u), (acc_i, out_i)):
        for t in range(rpt // ZB):
            r0 = s * rpt + t * ZB
            pltpu.sync_copy(accr.at[pl.ds(r0, ZB)],
                            outr.at[pl.ds(c * NSLOT + r0, ZB)])


BB = 4096 // NW


@functools.partial(
    pl.kernel,
    out_type=jax.ShapeDtypeStruct((4096,), jnp.float32),
    mesh=_mesh,
    compiler_params=pltpu.CompilerParams(use_tc_tiling_on_sc=False, needs_layout_passes=False),
    scratch_types=(
        [pltpu.VMEM((BB,), jnp.int32) for _ in range(7)]
        + [pltpu.VMEM((BB,), jnp.int32) for _ in range(6)]
        + [pltpu.VMEM((BB, D), jnp.float32) for _ in range(12)]
        + [pltpu.VMEM((BB,), jnp.float32) for _ in range(2)]
        + [pltpu.VMEM((LANES,), jnp.float32), pltpu.VMEM((BB,), jnp.float32)]
    ),
)
def _decode(uid_hbm, iid_hbm, cid_hbm, cf0, cf1, cf2, cf3,
            eu_hbm, u1_hbm, u2p_hbm, ei_hbm, i1_hbm, i2p_hbm,
            mapu_hbm, mapi_hbm,
            cfe_hbm, ie_hbm, ub_hbm, ib_hbm, gb_hbm, out,
            ub, ib, cb, cm0, cm1, cm2, cm3,
            su, si, ju0, ju1, ji0, ji1,
            bEu, bU1, bU2a, bU2b, bEi, bI1, bI2a, bI2b, r2, r3, r4, r5,
            bub, bib, bgb, outb):
    w = _wid()
    base = w * BB
    pltpu.sync_copy(uid_hbm.at[pl.ds(base, BB)], ub)
    pltpu.sync_copy(iid_hbm.at[pl.ds(base, BB)], ib)
    pltpu.sync_copy(cid_hbm.at[pl.ds(base, BB)], cb)
    pltpu.sync_copy(cf0.at[cb], cm0)
    pltpu.sync_copy(cf1.at[cb], cm1)
    pltpu.sync_copy(cf2.at[cb], cm2)
    pltpu.sync_copy(cf3.at[cb], cm3)
    pltpu.sync_copy(mapu_hbm.at[ub], su)
    pltpu.sync_copy(mapi_hbm.at[ib], si)

    @pl.loop(0, BB // LANES)
    def _ji(g):
        sl = pl.ds(g * LANES, LANES)
        vu = su[sl] - 1
        ju0[sl] = vu
        ju1[sl] = vu + NSLOT
        vi = si[sl] - 1
        ji0[sl] = vi
        ji1[sl] = vi + NSLOT

    pltpu.sync_copy(eu_hbm.at[ub], bEu)
    pltpu.sync_copy(u1_hbm.at[ub], bU1)
    pltpu.sync_copy(u2p_hbm.at[ju0], bU2a)
    pltpu.sync_copy(u2p_hbm.at[ju1], bU2b)
    pltpu.sync_copy(ei_hbm.at[ib], bEi)
    pltpu.sync_copy(i1_hbm.at[ib], bI1)
    pltpu.sync_copy(i2p_hbm.at[ji0], bI2a)
    pltpu.sync_copy(i2p_hbm.at[ji1], bI2b)
    pltpu.sync_copy(cfe_hbm.at[cm0], r2)
    pltpu.sync_copy(cfe_hbm.at[cm1], r3)
    pltpu.sync_copy(cfe_hbm.at[cm2], r4)
    pltpu.sync_copy(ie_hbm.at[cm3], r5)
    pltpu.sync_copy(ub_hbm.at[ub], bub)
    pltpu.sync_copy(ib_hbm.at[ib], bib)
    pltpu.sync_copy(gb_hbm, bgb)

    @pl.loop(0, BB // LANES)
    def _grp(g):
        r0 = g * LANES
        lanei = lax.iota(jnp.int32, LANES)
        res = jnp.zeros((LANES,), jnp.float32)
        for j in range(LANES):
            r = r0 + j
            tv = jnp.zeros((LANES,), jnp.float32)
            for k in range(D // LANES):
                sl = pl.ds(k * LANES, LANES)
                fu = (bEu[r, sl] + 0.5 * bU1[r, sl]
                      + 0.25 * (bU2a[r, sl] + bU2b[r, sl]))
                fi = (bEi[r, sl] + 0.5 * bI1[r, sl]
                      + 0.25 * (bI2a[r, sl] + bI2b[r, sl]))
                a = r2[r, sl]
                b = r3[r, sl]
                cc = r4[r, sl]
                dd = r5[r, sl]
                ssum = fu + fi + a + b + cc + dd
                sq = (fu * fu + fi * fi + a * a + b * b + cc * cc + dd * dd)
                tv = tv + (ssum * ssum - sq)
            tot = jnp.sum(tv)
            res = jnp.where(lanei == j, tot, res)
        res = (0.5 * res + bub[pl.ds(r0, LANES)] + bib[pl.ds(r0, LANES)]
               + bgb[pl.ds(0, LANES)])
        outb[pl.ds(r0, LANES)] = res

    pltpu.sync_copy(outb, out.at[pl.ds(base, BB)])


def _pad_rows(a, n):
    return jnp.concatenate(
        [a, jnp.zeros((n - a.shape[0],) + a.shape[1:], a.dtype)], axis=0)


def kernel(user_embeddings, item_embeddings, user_feature_embeddings,
           item_feature_embeddings, context_feature_embeddings,
           user_bias, item_bias, global_bias,
           user_id, item_id, context_id,
           user_feature_mat, item_feature_mat, context_feature_mat,
           insts2userid, insts2itemid, insts2contextid):
    ue_p = _pad_rows(user_embeddings, UPAD)
    ie_p = _pad_rows(item_embeddings, UPAD)
    ufm_p = _pad_rows(user_feature_mat, UPAD)
    ifm_p = _pad_rows(item_feature_mat, UPAD)
    cfm_p = _pad_rows(context_feature_mat, CPAD)
    uf = [ufm_p[:, j] + 0 for j in range(3)]
    if_ = [ifm_p[:, j] + 0 for j in range(3)]
    cf = [cfm_p[:, j] + 0 for j in range(4)]

    npad = IPAD - NINST
    uid_p = jnp.concatenate([insts2userid, jnp.full((npad,), SENT, jnp.int32)])
    iid_p = jnp.concatenate([insts2itemid, jnp.full((npad,), SENT, jnp.int32)])
    cid_p = jnp.concatenate([insts2contextid, jnp.zeros((npad,), jnp.int32)])

    C, C_bf = _pool_ctx(cf[0], cf[1], cf[2], cf[3],
                        context_feature_embeddings, context_feature_embeddings,
                        context_feature_embeddings, ie_p)
    EU, EU_bf = _pool_enc_u(ue_p, uf[0], uf[1], uf[2],
                          user_feature_embeddings, user_feature_embeddings,
                          user_feature_embeddings)
    EI, EI_bf = _pool_enc_i(ie_p, if_[0], if_[1], if_[2],
                          item_feature_embeddings, item_feature_embeddings,
                          item_feature_embeddings)

    U1 = _seg(uid_p, iid_p, cid_p, EI_bf, C_bf)
    I1 = _seg(iid_p, uid_p, cid_p, EU_bf, C_bf)

    maps = _slots(jnp.stack([user_id, item_id]))
    map_u = maps[0] + 0
    map_i = maps[1] + 0
    U2P, I2P = _seg2(uid_p, iid_p, cid_p, map_u, map_i, U1, I1, C)

    ub_flat = user_bias[:, 0] + 0
    ib_flat = item_bias[:, 0] + 0
    gb16 = jnp.broadcast_to(global_bias[0, 0], (LANES,)) + 0

    pred = _decode(user_id, item_id, context_id, cf[0], cf[1], cf[2], cf[3],
                   EU, U1, U2P, EI, I1, I2P, map_u, map_i,
                   context_feature_embeddings, ie_p,
                   ub_flat, ib_flat, gb16)
    return pred.reshape(4096, 1)
